# initial kernel scaffold (unmeasured)
import functools

import jax
import jax.numpy as jnp
from jax import lax
from jax.experimental import pallas as pl
from jax.experimental.pallas import tpu as pltpu

N = 8
B, S, D = 4, 256, 4096
H, Dh, Dr = 32, 128, 64
HL = H // N
KC = HL * Dh
QRC = HL * Dr
M = B * S
RC = M // N
SCALE = (Dh + Dr) ** -0.5
BF = jnp.bfloat16


def _partials_body(x_ref, wdkv_ref, wuk_ref, wuv_ref, kp_ref, vp_ref):
    c = jnp.dot(x_ref[...], wdkv_ref[...],
                preferred_element_type=jnp.float32).astype(BF)
    for k in range(N):
        sl = slice(k * KC, (k + 1) * KC)
        kp_ref[k] = jnp.dot(c, wuk_ref[:, sl],
                            preferred_element_type=jnp.float32).astype(BF)
        vp_ref[k] = jnp.dot(c, wuv_ref[:, sl],
                            preferred_element_type=jnp.float32).astype(BF)


def _partials(xb, wdkv, wuk, wuv):
    return pl.pallas_call(
        _partials_body,
        out_shape=[jax.ShapeDtypeStruct((N, M, KC), BF),
                   jax.ShapeDtypeStruct((N, M, KC), BF)],
        in_specs=[pl.BlockSpec(memory_space=pltpu.VMEM)] * 4,
        out_specs=[pl.BlockSpec(memory_space=pltpu.VMEM)] * 2,
    )(xb, wdkv, wuk, wuv)


def _ring_barrier(left, right):
    sem = pltpu.get_barrier_semaphore()
    for nbr in (left, right):
        pl.semaphore_signal(sem, inc=1, device_id=(nbr,),
                            device_id_type=pl.DeviceIdType.MESH)
    pl.semaphore_wait(sem, 2)


def _rs_kv_body(kp_ref, vp_ref, kj_ref, vj_ref, buf, send_sems, recv_sems):
    i = lax.axis_index("i")
    left = (i - 1) % N
    right = (i + 1) % N
    _ring_barrier(left, right)

    for s in range(N - 1):
        c_send = (i - s) % N
        src = N - 1 if s == 0 else s - 1
        if s == 0:
            buf[src, :, :KC] = kp_ref[c_send]
            buf[src, :, KC:] = vp_ref[c_send]
        else:
            buf[src, :, :KC] = buf[src, :, :KC] + kp_ref[c_send]
            buf[src, :, KC:] = buf[src, :, KC:] + vp_ref[c_send]
        rdma = pltpu.make_async_remote_copy(
            src_ref=buf.at[src],
            dst_ref=buf.at[s],
            send_sem=send_sems.at[s],
            recv_sem=recv_sems.at[s],
            device_id=(right,),
            device_id_type=pl.DeviceIdType.MESH,
        )
        rdma.start()
        rdma.wait()

    blk = (i + 1) % N
    kj_ref[...] = buf[N - 2, :, :KC] + kp_ref[blk]
    vj_ref[...] = buf[N - 2, :, KC:] + vp_ref[blk]


def _rs_kv(kp, vp):
    return pl.pallas_call(
        _rs_kv_body,
        out_shape=[jax.ShapeDtypeStruct((M, KC), BF),
                   jax.ShapeDtypeStruct((M, KC), BF)],
        in_specs=[pl.BlockSpec(memory_space=pltpu.VMEM)] * 2,
        out_specs=[pl.BlockSpec(memory_space=pltpu.VMEM)] * 2,
        scratch_shapes=[
            pltpu.VMEM((N, M, 2 * KC), BF),
            pltpu.SemaphoreType.DMA((N - 1,)),
            pltpu.SemaphoreType.DMA((N - 1,)),
        ],
        compiler_params=pltpu.CompilerParams(collective_id=0),
    )(kp, vp)


def _qproj_body(x_ref, wq_ref, wqr_ref, wkr_ref, q_ref, qr_ref, kr_ref):
    x = x_ref[...]
    q_ref[...] = jnp.dot(x, wq_ref[...],
                         preferred_element_type=jnp.float32).astype(BF)
    qr_ref[...] = jnp.dot(x, wqr_ref[...],
                          preferred_element_type=jnp.float32).astype(BF)
    kr_ref[...] = jnp.dot(x, wkr_ref[...],
                          preferred_element_type=jnp.float32).astype(BF)


def _qproj(xb, wq_j, wqr_j, wkr):
    return pl.pallas_call(
        _qproj_body,
        out_shape=[jax.ShapeDtypeStruct((M, KC), BF),
                   jax.ShapeDtypeStruct((M, QRC), BF),
                   jax.ShapeDtypeStruct((M, Dr), BF)],
        in_specs=[pl.BlockSpec(memory_space=pltpu.VMEM)] * 4,
        out_specs=[pl.BlockSpec(memory_space=pltpu.VMEM)] * 3,
    )(xb, wq_j, wqr_j, wkr)


def _attn_body(q_ref, k_ref, v_ref, qr_ref, kr_ref, o_ref):
    q = q_ref[0]
    k = k_ref[0]
    v = v_ref[0]
    qr = qr_ref[0]
    kr = kr_ref[0]
    nums = (((1,), (1,)), ((), ()))
    s = (lax.dot_general(q, k, nums, preferred_element_type=jnp.float32)
         + lax.dot_general(qr, kr, nums, preferred_element_type=jnp.float32)
         ) * SCALE
    m = jnp.max(s, axis=-1, keepdims=True)
    e = jnp.exp(s - m)
    p = (e / jnp.sum(e, axis=-1, keepdims=True)).astype(BF)
    o_ref[0] = jnp.dot(p, v, preferred_element_type=jnp.float32).astype(BF)


def _attention(q4, k4, v4, qr4, kr3):
    return pl.pallas_call(
        _attn_body,
        grid=(B * HL,),
        out_shape=jax.ShapeDtypeStruct((B * HL, S, Dh), BF),
        in_specs=[
            pl.BlockSpec((1, S, Dh), lambda p: (p, 0, 0)),
            pl.BlockSpec((1, S, Dh), lambda p: (p, 0, 0)),
            pl.BlockSpec((1, S, Dh), lambda p: (p, 0, 0)),
            pl.BlockSpec((1, S, Dr), lambda p: (p, 0, 0)),
            pl.BlockSpec((1, S, Dr), lambda p: (p // HL, 0, 0)),
        ],
        out_specs=pl.BlockSpec((1, S, Dh), lambda p: (p, 0, 0)),
    )(q4, k4, v4, qr4, kr3)


def _outpart_body(o_ref, wo_ref, out_ref):
    out_ref[...] = jnp.dot(o_ref[...], wo_ref[...],
                           preferred_element_type=jnp.float32).astype(BF)


def _outpart(oj, wo_j):
    return pl.pallas_call(
        _outpart_body,
        out_shape=jax.ShapeDtypeStruct((M, D), BF),
        in_specs=[pl.BlockSpec(memory_space=pltpu.VMEM)] * 2,
        out_specs=pl.BlockSpec(memory_space=pltpu.VMEM),
    )(oj, wo_j)


def _ar_body(p_ref, out_ref, rs_buf, ag_buf,
             rs_send, rs_recv, ag_send, ag_recv):
    i = lax.axis_index("i")
    left = (i - 1) % N
    right = (i + 1) % N
    _ring_barrier(left, right)

    for s in range(N - 1):
        c_send = (i - s) % N
        src = N - 1 if s == 0 else s - 1
        if s == 0:
            rs_buf[src] = p_ref[c_send]
        else:
            rs_buf[src] = rs_buf[src] + p_ref[c_send]
        rdma = pltpu.make_async_remote_copy(
            src_ref=rs_buf.at[src],
            dst_ref=rs_buf.at[s],
            send_sem=rs_send.at[s],
            recv_sem=rs_recv.at[s],
            device_id=(right,),
            device_id_type=pl.DeviceIdType.MESH,
        )
        rdma.start()
        rdma.wait()

    blk = (i + 1) % N
    mine = rs_buf[N - 2] + p_ref[blk]
    ag_buf[N - 1] = mine
    out_ref[blk] = mine.astype(jnp.float32)

    for s in range(N - 1):
        src = N - 1 if s == 0 else s - 1
        rdma = pltpu.make_async_remote_copy(
            src_ref=ag_buf.at[src],
            dst_ref=ag_buf.at[s],
            send_sem=ag_send.at[s],
            recv_sem=ag_recv.at[s],
            device_id=(right,),
            device_id_type=pl.DeviceIdType.MESH,
        )
        rdma.start()
        rdma.wait()
        out_ref[(i - s) % N] = ag_buf[s].astype(jnp.float32)


def _allreduce_out(p8):
    return pl.pallas_call(
        _ar_body,
        out_shape=jax.ShapeDtypeStruct((N, RC, D), jnp.float32),
        in_specs=[pl.BlockSpec(memory_space=pltpu.VMEM)],
        out_specs=pl.BlockSpec(memory_space=pltpu.VMEM),
        scratch_shapes=[
            pltpu.VMEM((N, RC, D), BF),
            pltpu.VMEM((N, RC, D), BF),
            pltpu.SemaphoreType.DMA((N - 1,)),
            pltpu.SemaphoreType.DMA((N - 1,)),
            pltpu.SemaphoreType.DMA((N - 1,)),
            pltpu.SemaphoreType.DMA((N - 1,)),
        ],
        compiler_params=pltpu.CompilerParams(collective_id=1),
    )(p8)


def kernel(x, Wdkv, Wuk, Wuv, Wq, Wqr, Wkr, Wo):
    i = lax.axis_index("i")
    blk = (i + 1) % N

    xb = x.reshape(M, D).astype(BF)
    kp, vp = _partials(xb, Wdkv.astype(BF), Wuk.astype(BF), Wuv.astype(BF))
    kj, vj = _rs_kv(kp, vp)

    wq_j = lax.dynamic_slice(Wq, (0, blk * KC), (D, KC)).astype(BF)
    wqr_j = lax.dynamic_slice(Wqr, (0, blk * QRC), (D, QRC)).astype(BF)
    wo_j = lax.dynamic_slice(Wo, (blk * KC, 0), (KC, D)).astype(BF)
    q, qr, kr = _qproj(xb, wq_j, wqr_j, Wkr.astype(BF))

    def to_heads(t, d):
        return t.reshape(B, S, HL, d).transpose(0, 2, 1, 3).reshape(B * HL, S, d)

    o4 = _attention(to_heads(q, Dh), to_heads(kj, Dh), to_heads(vj, Dh),
                    to_heads(qr, Dr), kr.reshape(B, S, Dr))
    oj = o4.reshape(B, HL, S, Dh).transpose(0, 2, 1, 3).reshape(M, KC)

    p8 = _outpart(oj, wo_j).reshape(N, RC, D)
    return _allreduce_out(p8).reshape(B, S, D)


# baseline (device time: 442286 ns/iter reference)
import jax
import jax.numpy as jnp
from jax import lax
from jax.experimental import pallas as pl
from jax.experimental.pallas import tpu as pltpu

N = 8
B, S, D = 4, 256, 4096
H, Dh, Dr = 32, 128, 64
HL = H // N
KC = HL * Dh
QRC = HL * Dr
M = B * S
RC = M // N
SCALE = (Dh + Dr) ** -0.5
BF = jnp.bfloat16
VMEM = pltpu.VMEM


def _latent_body(x_ref, w_ref, c_ref):
    c_ref[...] = jnp.dot(x_ref[...], w_ref[...],
                         preferred_element_type=jnp.float32).astype(BF)


def _latent(xb, wdkv):
    return pl.pallas_call(
        _latent_body,
        out_shape=jax.ShapeDtypeStruct((M, D // 32), BF),
        in_specs=[pl.BlockSpec(memory_space=VMEM)] * 2,
        out_specs=pl.BlockSpec(memory_space=VMEM),
    )(xb, wdkv)


def _expand_body(c_ref, w_ref, out_ref):
    c = c_ref[...]
    for k in range(N):
        out_ref[k] = jnp.dot(c, w_ref[:, k * KC:(k + 1) * KC],
                             preferred_element_type=jnp.float32).astype(BF)


def _expand(c, w):
    return pl.pallas_call(
        _expand_body,
        out_shape=jax.ShapeDtypeStruct((N, M, KC), BF),
        in_specs=[pl.BlockSpec(memory_space=VMEM)] * 2,
        out_specs=pl.BlockSpec(memory_space=VMEM),
    )(c, w)


def _qproj_body(x_ref, wq_ref, wqr_ref, wkr_ref, q_ref, qr_ref, kr_ref):
    x = x_ref[...]
    q_ref[...] = jnp.dot(x, wq_ref[...],
                         preferred_element_type=jnp.float32).astype(BF)
    qr_ref[...] = jnp.dot(x, wqr_ref[...],
                          preferred_element_type=jnp.float32).astype(BF)
    kr_ref[...] = jnp.dot(x, wkr_ref[...],
                          preferred_element_type=jnp.float32).astype(BF)


def _qproj(xb, wq_j, wqr_j, wkr):
    return pl.pallas_call(
        _qproj_body,
        out_shape=[jax.ShapeDtypeStruct((M, KC), BF),
                   jax.ShapeDtypeStruct((M, QRC), BF),
                   jax.ShapeDtypeStruct((M, Dr), BF)],
        in_specs=[pl.BlockSpec(memory_space=VMEM)] * 4,
        out_specs=[pl.BlockSpec(memory_space=VMEM)] * 3,
    )(xb, wq_j, wqr_j, wkr)


def _attn_body(q_ref, k_ref, v_ref, qr_ref, kr_ref, o_ref):
    q = q_ref[0]
    k = k_ref[0]
    v = v_ref[0]
    qr = qr_ref[0]
    kr = kr_ref[0]
    nums = (((1,), (1,)), ((), ()))
    s = (lax.dot_general(q, k, nums, preferred_element_type=jnp.float32)
         + lax.dot_general(qr, kr, nums, preferred_element_type=jnp.float32)
         ) * SCALE
    m = jnp.max(s, axis=-1, keepdims=True)
    e = jnp.exp(s - m)
    p = (e / jnp.sum(e, axis=-1, keepdims=True)).astype(BF)
    o_ref[0] = jnp.dot(p, v, preferred_element_type=jnp.float32).astype(BF)


def _attention(q4, k4, v4, qr4, kr3):
    return pl.pallas_call(
        _attn_body,
        grid=(B * HL,),
        out_shape=jax.ShapeDtypeStruct((B * HL, S, Dh), BF),
        in_specs=[
            pl.BlockSpec((1, S, Dh), lambda p: (p, 0, 0)),
            pl.BlockSpec((1, S, Dh), lambda p: (p, 0, 0)),
            pl.BlockSpec((1, S, Dh), lambda p: (p, 0, 0)),
            pl.BlockSpec((1, S, Dr), lambda p: (p, 0, 0)),
            pl.BlockSpec((1, S, Dr), lambda p: (p // HL, 0, 0)),
        ],
        out_specs=pl.BlockSpec((1, S, Dh), lambda p: (p, 0, 0)),
    )(q4, k4, v4, qr4, kr3)


def _outpart_body(o_ref, wo_ref, out_ref):
    out_ref[...] = jnp.dot(o_ref[...], wo_ref[...],
                           preferred_element_type=jnp.float32).astype(BF)


def _outpart(oj, wo_j):
    return pl.pallas_call(
        _outpart_body,
        out_shape=jax.ShapeDtypeStruct((M, D), BF),
        in_specs=[pl.BlockSpec(memory_space=VMEM)] * 2,
        out_specs=pl.BlockSpec(memory_space=VMEM),
    )(oj, wo_j)


def _ring_barrier(left, right):
    sem = pltpu.get_barrier_semaphore()
    for nbr in (left, right):
        pl.semaphore_signal(sem, inc=1, device_id=(nbr,),
                            device_id_type=pl.DeviceIdType.MESH)
    pl.semaphore_wait(sem, 2)


def _rs_body(p_ref, out_ref, buf, send_sems, recv_sems):
    i = lax.axis_index("i")
    left = (i - 1) % N
    right = (i + 1) % N
    _ring_barrier(left, right)

    for s in range(N - 1):
        c_send = (i - s) % N
        src = N - 1 if s == 0 else s - 1
        if s == 0:
            buf[src] = p_ref[c_send]
        else:
            buf[src] = buf[src] + p_ref[c_send]
        rdma = pltpu.make_async_remote_copy(
            src_ref=buf.at[src],
            dst_ref=buf.at[s],
            send_sem=send_sems.at[s],
            recv_sem=recv_sems.at[s],
            device_id=(right,),
            device_id_type=pl.DeviceIdType.MESH,
        )
        rdma.start()
        rdma.wait()

    out_ref[...] = buf[N - 2] + p_ref[(i + 1) % N]


def _ring_rs(p, collective_id):
    _, r, c = p.shape
    return pl.pallas_call(
        _rs_body,
        out_shape=jax.ShapeDtypeStruct((r, c), BF),
        in_specs=[pl.BlockSpec(memory_space=VMEM)],
        out_specs=pl.BlockSpec(memory_space=VMEM),
        scratch_shapes=[
            VMEM((N, r, c), BF),
            pltpu.SemaphoreType.DMA((N - 1,)),
            pltpu.SemaphoreType.DMA((N - 1,)),
        ],
        compiler_params=pltpu.CompilerParams(collective_id=collective_id),
    )(p)


def _ag_body(mine_ref, out_ref, buf, send_sems, recv_sems):
    i = lax.axis_index("i")
    left = (i - 1) % N
    right = (i + 1) % N
    _ring_barrier(left, right)

    buf[N - 1] = mine_ref[...]
    out_ref[(i + 1) % N] = mine_ref[...].astype(jnp.float32)
    for s in range(N - 1):
        src = N - 1 if s == 0 else s - 1
        rdma = pltpu.make_async_remote_copy(
            src_ref=buf.at[src],
            dst_ref=buf.at[s],
            send_sem=send_sems.at[s],
            recv_sem=recv_sems.at[s],
            device_id=(right,),
            device_id_type=pl.DeviceIdType.MESH,
        )
        rdma.start()
        rdma.wait()
        out_ref[(i - s) % N] = buf[s].astype(jnp.float32)


def _ring_ag(mine, collective_id):
    r, c = mine.shape
    return pl.pallas_call(
        _ag_body,
        out_shape=jax.ShapeDtypeStruct((N, r, c), jnp.float32),
        in_specs=[pl.BlockSpec(memory_space=VMEM)],
        out_specs=pl.BlockSpec(memory_space=VMEM),
        scratch_shapes=[
            VMEM((N, r, c), BF),
            pltpu.SemaphoreType.DMA((N - 1,)),
            pltpu.SemaphoreType.DMA((N - 1,)),
        ],
        compiler_params=pltpu.CompilerParams(collective_id=collective_id),
    )(mine)


def kernel(x, Wdkv, Wuk, Wuv, Wq, Wqr, Wkr, Wo):
    i = lax.axis_index("i")
    blk = (i + 1) % N

    xb = x.reshape(M, D).astype(BF)
    c = _latent(xb, Wdkv.astype(BF))
    kp = _expand(c, Wuk.astype(BF))
    vp = _expand(c, Wuv.astype(BF))
    kj = _ring_rs(kp, 0)
    vj = _ring_rs(vp, 1)

    wq_j = lax.dynamic_slice(Wq, (0, blk * KC), (D, KC)).astype(BF)
    wqr_j = lax.dynamic_slice(Wqr, (0, blk * QRC), (D, QRC)).astype(BF)
    wo_j = lax.dynamic_slice(Wo, (blk * KC, 0), (KC, D)).astype(BF)
    q, qr, kr = _qproj(xb, wq_j, wqr_j, Wkr.astype(BF))

    def to_heads(t, d):
        return t.reshape(B, S, HL, d).transpose(0, 2, 1, 3).reshape(B * HL, S, d)

    o4 = _attention(to_heads(q, Dh), to_heads(kj, Dh), to_heads(vj, Dh),
                    to_heads(qr, Dr), kr.reshape(B, S, Dr))
    oj = o4.reshape(B, HL, S, Dh).transpose(0, 2, 1, 3).reshape(M, KC)

    p8 = _outpart(oj, wo_j).reshape(N, RC, D)
    mine = _ring_rs(p8, 2)
    out = _ring_ag(mine, 3)
    return out.reshape(B, S, D)


# device time: 286701 ns/iter; 1.5427x vs baseline; 1.5427x over previous
import jax
import jax.numpy as jnp
from jax import lax
from jax.experimental import pallas as pl
from jax.experimental.pallas import tpu as pltpu

N = 8
B, S, D = 4, 256, 4096
H, Dh, Dr = 32, 128, 64
HL = H // N
KC = HL * Dh
QRC = HL * Dr
M = B * S
RC = M // N
SCALE = (Dh + Dr) ** -0.5
BF = jnp.bfloat16
VMEM = pltpu.VMEM


def _latent_body(x_ref, w_ref, c_ref):
    c_ref[...] = jnp.dot(x_ref[...], w_ref[...],
                         preferred_element_type=jnp.float32).astype(BF)


def _latent(xb, wdkv):
    return pl.pallas_call(
        _latent_body,
        out_shape=jax.ShapeDtypeStruct((M, D // 32), BF),
        in_specs=[pl.BlockSpec(memory_space=VMEM)] * 2,
        out_specs=pl.BlockSpec(memory_space=VMEM),
    )(xb, wdkv)


def _kvq_body(c_ref, wukb_ref, wuvb_ref, x_ref, wq_ref, wqr_ref, wkr_ref,
              q_ref, qr_ref, kr_ref, kj_ref, vj_ref,
              call_s, wka_s, wva_s, cs, cr, wks, wkrv, wvs, wvrv):
    i = lax.axis_index("i")
    blk = (i + 1) % N

    bsem = pltpu.get_barrier_semaphore()
    for d in range(1, N):
        pl.semaphore_signal(bsem, inc=1, device_id=((i + d) % N,),
                            device_id_type=pl.DeviceIdType.MESH)
    pl.semaphore_wait(bsem, N - 1)

    descs = []
    for d in range(1, N):
        tgt = (i + d) % N
        blk_tgt = (tgt + 1) % N
        for src, dst, ssem, rsem in (
            (c_ref, call_s, cs, cr),
            (wukb_ref.at[blk_tgt], wka_s, wks, wkrv),
            (wuvb_ref.at[blk_tgt], wva_s, wvs, wvrv),
        ):
            rdma = pltpu.make_async_remote_copy(
                src_ref=src,
                dst_ref=dst.at[d - 1],
                send_sem=ssem.at[d - 1],
                recv_sem=rsem.at[d - 1],
                device_id=(tgt,),
                device_id_type=pl.DeviceIdType.MESH,
            )
            rdma.start()
            descs.append(rdma)

    call_s[N - 1] = c_ref[...]
    wka_s[N - 1] = wukb_ref[blk]
    wva_s[N - 1] = wuvb_ref[blk]
    x = x_ref[...]
    q_ref[...] = jnp.dot(x, wq_ref[...],
                         preferred_element_type=jnp.float32).astype(BF)
    qr_ref[...] = jnp.dot(x, wqr_ref[...],
                          preferred_element_type=jnp.float32).astype(BF)
    kr_ref[...] = jnp.dot(x, wkr_ref[...],
                          preferred_element_type=jnp.float32).astype(BF)

    for rdma in descs:
        rdma.wait()

    k = jnp.zeros((M, KC), jnp.float32)
    v = jnp.zeros((M, KC), jnp.float32)
    for g in range(N):
        cg = call_s[g]
        k = k + jnp.dot(cg, wka_s[g], preferred_element_type=jnp.float32)
        v = v + jnp.dot(cg, wva_s[g], preferred_element_type=jnp.float32)
    kj_ref[...] = k.astype(BF)
    vj_ref[...] = v.astype(BF)

    def _exit(sem):
        for d in range(1, N):
            pl.semaphore_signal(sem, inc=1, device_id=((i + d) % N,),
                                device_id_type=pl.DeviceIdType.MESH)
        pl.semaphore_wait(sem, N - 1)

    pl.run_scoped(_exit, pltpu.SemaphoreType.REGULAR)


def _kvq(c, wukb, wuvb, xb, wq_j, wqr_j, wkr):
    dc = D // 32
    return pl.pallas_call(
        _kvq_body,
        out_shape=[jax.ShapeDtypeStruct((M, KC), BF),
                   jax.ShapeDtypeStruct((M, QRC), BF),
                   jax.ShapeDtypeStruct((M, Dr), BF),
                   jax.ShapeDtypeStruct((M, KC), BF),
                   jax.ShapeDtypeStruct((M, KC), BF)],
        in_specs=[pl.BlockSpec(memory_space=VMEM)] * 7,
        out_specs=[pl.BlockSpec(memory_space=VMEM)] * 5,
        scratch_shapes=[
            VMEM((N, M, dc), BF),
            VMEM((N, dc, KC), BF),
            VMEM((N, dc, KC), BF),
            pltpu.SemaphoreType.DMA((N - 1,)),
            pltpu.SemaphoreType.DMA((N - 1,)),
            pltpu.SemaphoreType.DMA((N - 1,)),
            pltpu.SemaphoreType.DMA((N - 1,)),
            pltpu.SemaphoreType.DMA((N - 1,)),
            pltpu.SemaphoreType.DMA((N - 1,)),
        ],
        compiler_params=pltpu.CompilerParams(collective_id=0),
    )(c, wukb, wuvb, xb, wq_j, wqr_j, wkr)


def _attn_body(q_ref, k_ref, v_ref, qr_ref, kr_ref, o_ref):
    q = q_ref[0]
    k = k_ref[0]
    v = v_ref[0]
    qr = qr_ref[0]
    kr = kr_ref[0]
    nums = (((1,), (1,)), ((), ()))
    s = (lax.dot_general(q, k, nums, preferred_element_type=jnp.float32)
         + lax.dot_general(qr, kr, nums, preferred_element_type=jnp.float32)
         ) * SCALE
    m = jnp.max(s, axis=-1, keepdims=True)
    e = jnp.exp(s - m)
    p = (e / jnp.sum(e, axis=-1, keepdims=True)).astype(BF)
    o_ref[0] = jnp.dot(p, v, preferred_element_type=jnp.float32).astype(BF)


def _attention(q4, k4, v4, qr4, kr3):
    return pl.pallas_call(
        _attn_body,
        grid=(B * HL,),
        out_shape=jax.ShapeDtypeStruct((B * HL, S, Dh), BF),
        in_specs=[
            pl.BlockSpec((1, S, Dh), lambda p: (p, 0, 0)),
            pl.BlockSpec((1, S, Dh), lambda p: (p, 0, 0)),
            pl.BlockSpec((1, S, Dh), lambda p: (p, 0, 0)),
            pl.BlockSpec((1, S, Dr), lambda p: (p, 0, 0)),
            pl.BlockSpec((1, S, Dr), lambda p: (p // HL, 0, 0)),
        ],
        out_specs=pl.BlockSpec((1, S, Dh), lambda p: (p, 0, 0)),
    )(q4, k4, v4, qr4, kr3)


def _outpart_body(o_ref, wo_ref, out_ref):
    out_ref[...] = jnp.dot(o_ref[...], wo_ref[...],
                           preferred_element_type=jnp.float32).astype(BF)


def _outpart(oj, wo_j):
    return pl.pallas_call(
        _outpart_body,
        out_shape=jax.ShapeDtypeStruct((M, D), BF),
        in_specs=[pl.BlockSpec(memory_space=VMEM)] * 2,
        out_specs=pl.BlockSpec(memory_space=VMEM),
    )(oj, wo_j)


def _ring_barrier(left, right):
    sem = pltpu.get_barrier_semaphore()
    for nbr in (left, right):
        pl.semaphore_signal(sem, inc=1, device_id=(nbr,),
                            device_id_type=pl.DeviceIdType.MESH)
    pl.semaphore_wait(sem, 2)


def _rs_body(p_ref, out_ref, buf, send_sems, recv_sems):
    i = lax.axis_index("i")
    left = (i - 1) % N
    right = (i + 1) % N
    _ring_barrier(left, right)

    for s in range(N - 1):
        c_send = (i - s) % N
        src = N - 1 if s == 0 else s - 1
        if s == 0:
            buf[src] = p_ref[c_send]
        else:
            buf[src] = buf[src] + p_ref[c_send]
        rdma = pltpu.make_async_remote_copy(
            src_ref=buf.at[src],
            dst_ref=buf.at[s],
            send_sem=send_sems.at[s],
            recv_sem=recv_sems.at[s],
            device_id=(right,),
            device_id_type=pl.DeviceIdType.MESH,
        )
        rdma.start()
        rdma.wait()

    out_ref[...] = buf[N - 2] + p_ref[(i + 1) % N]


def _ring_rs(p, collective_id):
    _, r, c = p.shape
    return pl.pallas_call(
        _rs_body,
        out_shape=jax.ShapeDtypeStruct((r, c), BF),
        in_specs=[pl.BlockSpec(memory_space=VMEM)],
        out_specs=pl.BlockSpec(memory_space=VMEM),
        scratch_shapes=[
            VMEM((N, r, c), BF),
            pltpu.SemaphoreType.DMA((N - 1,)),
            pltpu.SemaphoreType.DMA((N - 1,)),
        ],
        compiler_params=pltpu.CompilerParams(collective_id=collective_id),
    )(p)


def _ag_body(mine_ref, out_ref, buf, send_sems, recv_sems):
    i = lax.axis_index("i")
    left = (i - 1) % N
    right = (i + 1) % N
    _ring_barrier(left, right)

    buf[N - 1] = mine_ref[...]
    out_ref[(i + 1) % N] = mine_ref[...].astype(jnp.float32)
    for s in range(N - 1):
        src = N - 1 if s == 0 else s - 1
        rdma = pltpu.make_async_remote_copy(
            src_ref=buf.at[src],
            dst_ref=buf.at[s],
            send_sem=send_sems.at[s],
            recv_sem=recv_sems.at[s],
            device_id=(right,),
            device_id_type=pl.DeviceIdType.MESH,
        )
        rdma.start()
        rdma.wait()
        out_ref[(i - s) % N] = buf[s].astype(jnp.float32)


def _ring_ag(mine, collective_id):
    r, c = mine.shape
    return pl.pallas_call(
        _ag_body,
        out_shape=jax.ShapeDtypeStruct((N, r, c), jnp.float32),
        in_specs=[pl.BlockSpec(memory_space=VMEM)],
        out_specs=pl.BlockSpec(memory_space=VMEM),
        scratch_shapes=[
            VMEM((N, r, c), BF),
            pltpu.SemaphoreType.DMA((N - 1,)),
            pltpu.SemaphoreType.DMA((N - 1,)),
        ],
        compiler_params=pltpu.CompilerParams(collective_id=collective_id),
    )(mine)


def kernel(x, Wdkv, Wuk, Wuv, Wq, Wqr, Wkr, Wo):
    i = lax.axis_index("i")
    blk = (i + 1) % N

    xb = x.reshape(M, D).astype(BF)
    c = _latent(xb, Wdkv.astype(BF))
    dc = Wdkv.shape[1]
    wukb = Wuk.astype(BF).reshape(dc, N, KC).transpose(1, 0, 2)
    wuvb = Wuv.astype(BF).reshape(dc, N, KC).transpose(1, 0, 2)

    wq_j = lax.dynamic_slice(Wq, (0, blk * KC), (D, KC)).astype(BF)
    wqr_j = lax.dynamic_slice(Wqr, (0, blk * QRC), (D, QRC)).astype(BF)
    wo_j = lax.dynamic_slice(Wo, (blk * KC, 0), (KC, D)).astype(BF)
    q, qr, kr, kj, vj = _kvq(c, wukb, wuvb, xb, wq_j, wqr_j, Wkr.astype(BF))

    def to_heads(t, d):
        return t.reshape(B, S, HL, d).transpose(0, 2, 1, 3).reshape(B * HL, S, d)

    o4 = _attention(to_heads(q, Dh), to_heads(kj, Dh), to_heads(vj, Dh),
                    to_heads(qr, Dr), kr.reshape(B, S, Dr))
    oj = o4.reshape(B, HL, S, Dh).transpose(0, 2, 1, 3).reshape(M, KC)

    p8 = _outpart(oj, wo_j).reshape(N, RC, D)
    mine = _ring_rs(p8, 2)
    out = _ring_ag(mine, 3)
    return out.reshape(B, S, D)


# device time: 204984 ns/iter; 2.1577x vs baseline; 1.3987x over previous
import jax
import jax.numpy as jnp
from jax import lax
from jax.experimental import pallas as pl
from jax.experimental.pallas import tpu as pltpu

N = 8
B, S, D = 4, 256, 4096
H, Dh, Dr = 32, 128, 64
HL = H // N
KC = HL * Dh
QRC = HL * Dr
M = B * S
RC = M // N
SCALE = (Dh + Dr) ** -0.5
BF = jnp.bfloat16
VMEM = pltpu.VMEM


def _latent_body(x_ref, w_ref, c_ref):
    c_ref[...] = jnp.dot(x_ref[...], w_ref[...],
                         preferred_element_type=jnp.float32).astype(BF)


def _latent(xb, wdkv):
    return pl.pallas_call(
        _latent_body,
        out_shape=jax.ShapeDtypeStruct((M, D // 32), BF),
        in_specs=[pl.BlockSpec(memory_space=VMEM)] * 2,
        out_specs=pl.BlockSpec(memory_space=VMEM),
    )(xb, wdkv)


def _kvq_body(c_ref, wukb_ref, wuvb_ref, x_ref, wq_ref, wqr_ref, wkr_ref,
              q_ref, qr_ref, kr_ref, kj_ref, vj_ref,
              call_s, wka_s, wva_s, cs, cr, wks, wkrv, wvs, wvrv):
    i = lax.axis_index("i")
    blk = (i + 1) % N

    bsem = pltpu.get_barrier_semaphore()
    for d in range(1, N):
        pl.semaphore_signal(bsem, inc=1, device_id=((i + d) % N,),
                            device_id_type=pl.DeviceIdType.MESH)
    pl.semaphore_wait(bsem, N - 1)

    descs = []
    for d in range(1, N):
        tgt = (i + d) % N
        blk_tgt = (tgt + 1) % N
        for src, dst, ssem, rsem in (
            (c_ref, call_s, cs, cr),
            (wukb_ref.at[blk_tgt], wka_s, wks, wkrv),
            (wuvb_ref.at[blk_tgt], wva_s, wvs, wvrv),
        ):
            rdma = pltpu.make_async_remote_copy(
                src_ref=src,
                dst_ref=dst.at[d - 1],
                send_sem=ssem.at[d - 1],
                recv_sem=rsem.at[d - 1],
                device_id=(tgt,),
                device_id_type=pl.DeviceIdType.MESH,
            )
            rdma.start()
            descs.append(rdma)

    call_s[N - 1] = c_ref[...]
    wka_s[N - 1] = wukb_ref[blk]
    wva_s[N - 1] = wuvb_ref[blk]
    x = x_ref[...]
    q_ref[...] = jnp.dot(x, wq_ref[...],
                         preferred_element_type=jnp.float32).astype(BF)
    qr_ref[...] = jnp.dot(x, wqr_ref[...],
                          preferred_element_type=jnp.float32).astype(BF)
    kr_ref[...] = jnp.dot(x, wkr_ref[...],
                          preferred_element_type=jnp.float32).astype(BF)

    for rdma in descs:
        rdma.wait()

    k = jnp.zeros((M, KC), jnp.float32)
    v = jnp.zeros((M, KC), jnp.float32)
    for g in range(N):
        cg = call_s[g]
        k = k + jnp.dot(cg, wka_s[g], preferred_element_type=jnp.float32)
        v = v + jnp.dot(cg, wva_s[g], preferred_element_type=jnp.float32)
    kj_ref[...] = k.astype(BF)
    vj_ref[...] = v.astype(BF)

    def _exit(sem):
        for d in range(1, N):
            pl.semaphore_signal(sem, inc=1, device_id=((i + d) % N,),
                                device_id_type=pl.DeviceIdType.MESH)
        pl.semaphore_wait(sem, N - 1)

    pl.run_scoped(_exit, pltpu.SemaphoreType.REGULAR)


def _kvq(c, wukb, wuvb, xb, wq_j, wqr_j, wkr):
    dc = D // 32
    return pl.pallas_call(
        _kvq_body,
        out_shape=[jax.ShapeDtypeStruct((M, KC), BF),
                   jax.ShapeDtypeStruct((M, QRC), BF),
                   jax.ShapeDtypeStruct((M, Dr), BF),
                   jax.ShapeDtypeStruct((M, KC), BF),
                   jax.ShapeDtypeStruct((M, KC), BF)],
        in_specs=[pl.BlockSpec(memory_space=VMEM)] * 7,
        out_specs=[pl.BlockSpec(memory_space=VMEM)] * 5,
        scratch_shapes=[
            VMEM((N, M, dc), BF),
            VMEM((N, dc, KC), BF),
            VMEM((N, dc, KC), BF),
            pltpu.SemaphoreType.DMA((N - 1,)),
            pltpu.SemaphoreType.DMA((N - 1,)),
            pltpu.SemaphoreType.DMA((N - 1,)),
            pltpu.SemaphoreType.DMA((N - 1,)),
            pltpu.SemaphoreType.DMA((N - 1,)),
            pltpu.SemaphoreType.DMA((N - 1,)),
        ],
        compiler_params=pltpu.CompilerParams(collective_id=0),
    )(c, wukb, wuvb, xb, wq_j, wqr_j, wkr)


def _attn_body(q_ref, k_ref, v_ref, qr_ref, kr_ref, o_ref):
    q = q_ref[0]
    k = k_ref[0]
    v = v_ref[0]
    qr = qr_ref[0]
    kr = kr_ref[0]
    nums = (((1,), (1,)), ((), ()))
    s = (lax.dot_general(q, k, nums, preferred_element_type=jnp.float32)
         + lax.dot_general(qr, kr, nums, preferred_element_type=jnp.float32)
         ) * SCALE
    m = jnp.max(s, axis=-1, keepdims=True)
    e = jnp.exp(s - m)
    p = (e / jnp.sum(e, axis=-1, keepdims=True)).astype(BF)
    o_ref[0] = jnp.dot(p, v, preferred_element_type=jnp.float32).astype(BF)


def _attention(q4, k4, v4, qr4, kr3):
    return pl.pallas_call(
        _attn_body,
        grid=(B * HL,),
        out_shape=jax.ShapeDtypeStruct((B * HL, S, Dh), BF),
        in_specs=[
            pl.BlockSpec((1, S, Dh), lambda p: (p, 0, 0)),
            pl.BlockSpec((1, S, Dh), lambda p: (p, 0, 0)),
            pl.BlockSpec((1, S, Dh), lambda p: (p, 0, 0)),
            pl.BlockSpec((1, S, Dr), lambda p: (p, 0, 0)),
            pl.BlockSpec((1, S, Dr), lambda p: (p // HL, 0, 0)),
        ],
        out_specs=pl.BlockSpec((1, S, Dh), lambda p: (p, 0, 0)),
    )(q4, k4, v4, qr4, kr3)


RH = RC // 2


def _outar_body(o_ref, wo_ref, out_ref, buf_r, buf_l,
                rs_r_s, rs_r_r, rs_l_s, rs_l_r,
                ag_r_s, ag_r_r, ag_l_s, ag_l_r):
    i = lax.axis_index("i")
    left = (i - 1) % N
    right = (i + 1) % N
    bsem = pltpu.get_barrier_semaphore()
    for nbr in (left, right):
        pl.semaphore_signal(bsem, inc=1, device_id=(nbr,),
                            device_id_type=pl.DeviceIdType.MESH)
    pl.semaphore_wait(bsem, 2)

    wo = wo_ref[...]

    def part(sub):
        o_rows = o_ref[pl.ds(sub * RH, RH), :]
        return jnp.dot(o_rows, wo, preferred_element_type=jnp.float32)

    buf_r[N - 1] = part(2 * i).astype(BF)
    buf_l[N - 1] = part(2 * i + 1).astype(BF)
    for s in range(N - 1):
        src = N - 1 if s == 0 else s - 1
        rd_r = pltpu.make_async_remote_copy(
            src_ref=buf_r.at[src], dst_ref=buf_r.at[s],
            send_sem=rs_r_s.at[s], recv_sem=rs_r_r.at[s],
            device_id=(right,), device_id_type=pl.DeviceIdType.MESH)
        rd_l = pltpu.make_async_remote_copy(
            src_ref=buf_l.at[src], dst_ref=buf_l.at[s],
            send_sem=rs_l_s.at[s], recv_sem=rs_l_r.at[s],
            device_id=(left,), device_id_type=pl.DeviceIdType.MESH)
        rd_r.start()
        rd_l.start()
        nxt_r = part(2 * ((i - s - 1) % N))
        nxt_l = part(2 * ((i + s + 1) % N) + 1)
        rd_r.wait()
        rd_l.wait()
        if s < N - 2:
            buf_r[s] = (buf_r[s] + nxt_r).astype(BF)
            buf_l[s] = (buf_l[s] + nxt_l).astype(BF)
        else:
            out_ref[2 * ((i + 1) % N)] = (buf_r[s] + nxt_r).astype(BF)
            out_ref[2 * ((i - 1) % N) + 1] = (buf_l[s] + nxt_l).astype(BF)

    for s in range(N - 1):
        sub_r = 2 * ((i + 1 - s) % N)
        sub_l = 2 * ((i - 1 + s) % N) + 1
        rd_r = pltpu.make_async_remote_copy(
            src_ref=out_ref.at[sub_r], dst_ref=out_ref.at[sub_r],
            send_sem=ag_r_s.at[s], recv_sem=ag_r_r.at[s],
            device_id=(right,), device_id_type=pl.DeviceIdType.MESH)
        rd_l = pltpu.make_async_remote_copy(
            src_ref=out_ref.at[sub_l], dst_ref=out_ref.at[sub_l],
            send_sem=ag_l_s.at[s], recv_sem=ag_l_r.at[s],
            device_id=(left,), device_id_type=pl.DeviceIdType.MESH)
        rd_r.start()
        rd_l.start()
        rd_r.wait()
        rd_l.wait()


def _outar(oj, wo_j):
    return pl.pallas_call(
        _outar_body,
        out_shape=jax.ShapeDtypeStruct((2 * N, RH, D), BF),
        in_specs=[pl.BlockSpec(memory_space=VMEM)] * 2,
        out_specs=pl.BlockSpec(memory_space=VMEM),
        scratch_shapes=[
            VMEM((N, RH, D), BF),
            VMEM((N, RH, D), BF),
        ] + [pltpu.SemaphoreType.DMA((N - 1,))] * 8,
        compiler_params=pltpu.CompilerParams(collective_id=1),
    )(oj, wo_j)


def kernel(x, Wdkv, Wuk, Wuv, Wq, Wqr, Wkr, Wo):
    i = lax.axis_index("i")
    blk = (i + 1) % N

    xb = x.reshape(M, D).astype(BF)
    c = _latent(xb, Wdkv.astype(BF))
    dc = Wdkv.shape[1]
    wukb = Wuk.astype(BF).reshape(dc, N, KC).transpose(1, 0, 2)
    wuvb = Wuv.astype(BF).reshape(dc, N, KC).transpose(1, 0, 2)

    wq_j = lax.dynamic_slice(Wq, (0, blk * KC), (D, KC)).astype(BF)
    wqr_j = lax.dynamic_slice(Wqr, (0, blk * QRC), (D, QRC)).astype(BF)
    wo_j = lax.dynamic_slice(Wo, (blk * KC, 0), (KC, D)).astype(BF)
    q, qr, kr, kj, vj = _kvq(c, wukb, wuvb, xb, wq_j, wqr_j, Wkr.astype(BF))

    def to_heads(t, d):
        return t.reshape(B, S, HL, d).transpose(0, 2, 1, 3).reshape(B * HL, S, d)

    o4 = _attention(to_heads(q, Dh), to_heads(kj, Dh), to_heads(vj, Dh),
                    to_heads(qr, Dr), kr.reshape(B, S, Dr))
    oj = o4.reshape(B, HL, S, Dh).transpose(0, 2, 1, 3).reshape(M, KC)

    out16 = _outar(oj, wo_j)
    return out16.reshape(M, D).astype(jnp.float32).reshape(B, S, D)


# device time: 203159 ns/iter; 2.1770x vs baseline; 1.0090x over previous
import jax
import jax.numpy as jnp
from jax import lax
from jax.experimental import pallas as pl
from jax.experimental.pallas import tpu as pltpu

N = 8
B, S, D = 4, 256, 4096
H, Dh, Dr = 32, 128, 64
HL = H // N
KC = HL * Dh
QRC = HL * Dr
M = B * S
RC = M // N
SCALE = (Dh + Dr) ** -0.5
BF = jnp.bfloat16
VMEM = pltpu.VMEM


def _kvq_body(x_ref, wdkv_ref, wukb_ref, wuvb_ref, wq_ref, wqr_ref, wkr_ref,
              q_ref, qr_ref, kr_ref, kj_ref, vj_ref,
              c_s, call_s, wka_s, wva_s, cs, cr, wks, wkrv, wvs, wvrv):
    i = lax.axis_index("i")
    blk = (i + 1) % N

    c_s[...] = jnp.dot(x_ref[...], wdkv_ref[...],
                       preferred_element_type=jnp.float32).astype(BF)

    bsem = pltpu.get_barrier_semaphore()
    for d in range(1, N):
        pl.semaphore_signal(bsem, inc=1, device_id=((i + d) % N,),
                            device_id_type=pl.DeviceIdType.MESH)
    pl.semaphore_wait(bsem, N - 1)

    descs = []
    for d in range(1, N):
        tgt = (i + d) % N
        blk_tgt = (tgt + 1) % N
        for src, dst, ssem, rsem in (
            (c_s, call_s, cs, cr),
            (wukb_ref.at[blk_tgt], wka_s, wks, wkrv),
            (wuvb_ref.at[blk_tgt], wva_s, wvs, wvrv),
        ):
            rdma = pltpu.make_async_remote_copy(
                src_ref=src,
                dst_ref=dst.at[d - 1],
                send_sem=ssem.at[d - 1],
                recv_sem=rsem.at[d - 1],
                device_id=(tgt,),
                device_id_type=pl.DeviceIdType.MESH,
            )
            rdma.start()
            descs.append(rdma)

    call_s[N - 1] = c_s[...]
    wka_s[N - 1] = wukb_ref[blk]
    wva_s[N - 1] = wuvb_ref[blk]
    x = x_ref[...]
    q_ref[...] = jnp.dot(x, wq_ref[...],
                         preferred_element_type=jnp.float32).astype(BF)
    qr_ref[...] = jnp.dot(x, wqr_ref[...],
                          preferred_element_type=jnp.float32).astype(BF)
    kr_ref[...] = jnp.dot(x, wkr_ref[...],
                          preferred_element_type=jnp.float32).astype(BF)

    for rdma in descs:
        rdma.wait()

    k = jnp.zeros((M, KC), jnp.float32)
    v = jnp.zeros((M, KC), jnp.float32)
    for g in range(N):
        cg = call_s[g]
        k = k + jnp.dot(cg, wka_s[g], preferred_element_type=jnp.float32)
        v = v + jnp.dot(cg, wva_s[g], preferred_element_type=jnp.float32)
    kj_ref[...] = k.astype(BF)
    vj_ref[...] = v.astype(BF)

    def _exit(sem):
        for d in range(1, N):
            pl.semaphore_signal(sem, inc=1, device_id=((i + d) % N,),
                                device_id_type=pl.DeviceIdType.MESH)
        pl.semaphore_wait(sem, N - 1)

    pl.run_scoped(_exit, pltpu.SemaphoreType.REGULAR)


def _kvq(xb, wdkv, wukb, wuvb, wq_j, wqr_j, wkr):
    dc = wdkv.shape[1]
    return pl.pallas_call(
        _kvq_body,
        out_shape=[jax.ShapeDtypeStruct((M, KC), BF),
                   jax.ShapeDtypeStruct((M, QRC), BF),
                   jax.ShapeDtypeStruct((M, Dr), BF),
                   jax.ShapeDtypeStruct((M, KC), BF),
                   jax.ShapeDtypeStruct((M, KC), BF)],
        in_specs=[pl.BlockSpec(memory_space=VMEM)] * 7,
        out_specs=[pl.BlockSpec(memory_space=VMEM)] * 5,
        scratch_shapes=[
            VMEM((M, dc), BF),
            VMEM((N, M, dc), BF),
            VMEM((N, dc, KC), BF),
            VMEM((N, dc, KC), BF),
            pltpu.SemaphoreType.DMA((N - 1,)),
            pltpu.SemaphoreType.DMA((N - 1,)),
            pltpu.SemaphoreType.DMA((N - 1,)),
            pltpu.SemaphoreType.DMA((N - 1,)),
            pltpu.SemaphoreType.DMA((N - 1,)),
            pltpu.SemaphoreType.DMA((N - 1,)),
        ],
        compiler_params=pltpu.CompilerParams(collective_id=0),
    )(xb, wdkv, wukb, wuvb, wq_j, wqr_j, wkr)


def _attn_body(q_ref, k_ref, v_ref, qr_ref, kr_ref, o_ref):
    kr = kr_ref[...]
    nums = (((1,), (1,)), ((), ()))
    for h in range(HL):
        hd = slice(h * Dh, (h + 1) * Dh)
        hr = slice(h * Dr, (h + 1) * Dr)
        s = (lax.dot_general(q_ref[:, hd], k_ref[:, hd], nums,
                             preferred_element_type=jnp.float32)
             + lax.dot_general(qr_ref[:, hr], kr, nums,
                               preferred_element_type=jnp.float32)) * SCALE
        m = jnp.max(s, axis=-1, keepdims=True)
        e = jnp.exp(s - m)
        p = (e / jnp.sum(e, axis=-1, keepdims=True)).astype(BF)
        o_ref[:, hd] = jnp.dot(p, v_ref[:, hd],
                               preferred_element_type=jnp.float32).astype(BF)


def _attention(q, kj, vj, qr, kr):
    return pl.pallas_call(
        _attn_body,
        grid=(B,),
        out_shape=jax.ShapeDtypeStruct((M, KC), BF),
        in_specs=[
            pl.BlockSpec((S, KC), lambda b: (b, 0)),
            pl.BlockSpec((S, KC), lambda b: (b, 0)),
            pl.BlockSpec((S, KC), lambda b: (b, 0)),
            pl.BlockSpec((S, QRC), lambda b: (b, 0)),
            pl.BlockSpec((S, Dr), lambda b: (b, 0)),
        ],
        out_specs=pl.BlockSpec((S, KC), lambda b: (b, 0)),
    )(q, kj, vj, qr, kr)


RH = RC // 2


def _outar_body(o_ref, wo_ref, out_ref, buf_r, buf_l,
                rs_r_s, rs_r_r, rs_l_s, rs_l_r,
                ag_r_s, ag_r_r, ag_l_s, ag_l_r):
    i = lax.axis_index("i")
    left = (i - 1) % N
    right = (i + 1) % N
    bsem = pltpu.get_barrier_semaphore()
    for nbr in (left, right):
        pl.semaphore_signal(bsem, inc=1, device_id=(nbr,),
                            device_id_type=pl.DeviceIdType.MESH)
    pl.semaphore_wait(bsem, 2)

    wo = wo_ref[...]

    def part(sub):
        o_rows = o_ref[pl.ds(sub * RH, RH), :]
        return jnp.dot(o_rows, wo, preferred_element_type=jnp.float32)

    buf_r[N - 1] = part(2 * i).astype(BF)
    buf_l[N - 1] = part(2 * i + 1).astype(BF)
    for s in range(N - 1):
        src = N - 1 if s == 0 else s - 1
        rd_r = pltpu.make_async_remote_copy(
            src_ref=buf_r.at[src], dst_ref=buf_r.at[s],
            send_sem=rs_r_s.at[s], recv_sem=rs_r_r.at[s],
            device_id=(right,), device_id_type=pl.DeviceIdType.MESH)
        rd_l = pltpu.make_async_remote_copy(
            src_ref=buf_l.at[src], dst_ref=buf_l.at[s],
            send_sem=rs_l_s.at[s], recv_sem=rs_l_r.at[s],
            device_id=(left,), device_id_type=pl.DeviceIdType.MESH)
        rd_r.start()
        rd_l.start()
        nxt_r = part(2 * ((i - s - 1) % N))
        nxt_l = part(2 * ((i + s + 1) % N) + 1)
        rd_r.wait()
        rd_l.wait()
        if s < N - 2:
            buf_r[s] = (buf_r[s] + nxt_r).astype(BF)
            buf_l[s] = (buf_l[s] + nxt_l).astype(BF)
        else:
            out_ref[2 * ((i + 1) % N)] = (buf_r[s] + nxt_r).astype(BF)
            out_ref[2 * ((i - 1) % N) + 1] = (buf_l[s] + nxt_l).astype(BF)

    for s in range(N - 1):
        sub_r = 2 * ((i + 1 - s) % N)
        sub_l = 2 * ((i - 1 + s) % N) + 1
        rd_r = pltpu.make_async_remote_copy(
            src_ref=out_ref.at[sub_r], dst_ref=out_ref.at[sub_r],
            send_sem=ag_r_s.at[s], recv_sem=ag_r_r.at[s],
            device_id=(right,), device_id_type=pl.DeviceIdType.MESH)
        rd_l = pltpu.make_async_remote_copy(
            src_ref=out_ref.at[sub_l], dst_ref=out_ref.at[sub_l],
            send_sem=ag_l_s.at[s], recv_sem=ag_l_r.at[s],
            device_id=(left,), device_id_type=pl.DeviceIdType.MESH)
        rd_r.start()
        rd_l.start()
        rd_r.wait()
        rd_l.wait()


def _outar(oj, wo_j):
    return pl.pallas_call(
        _outar_body,
        out_shape=jax.ShapeDtypeStruct((2 * N, RH, D), BF),
        in_specs=[pl.BlockSpec(memory_space=VMEM)] * 2,
        out_specs=pl.BlockSpec(memory_space=VMEM),
        scratch_shapes=[
            VMEM((N, RH, D), BF),
            VMEM((N, RH, D), BF),
        ] + [pltpu.SemaphoreType.DMA((N - 1,))] * 8,
        compiler_params=pltpu.CompilerParams(collective_id=1),
    )(oj, wo_j)


def kernel(x, Wdkv, Wuk, Wuv, Wq, Wqr, Wkr, Wo):
    i = lax.axis_index("i")
    blk = (i + 1) % N

    xb = x.reshape(M, D).astype(BF)
    dc = Wdkv.shape[1]
    wukb = Wuk.astype(BF).reshape(dc, N, KC).transpose(1, 0, 2)
    wuvb = Wuv.astype(BF).reshape(dc, N, KC).transpose(1, 0, 2)

    wq_j = lax.dynamic_slice(Wq, (0, blk * KC), (D, KC)).astype(BF)
    wqr_j = lax.dynamic_slice(Wqr, (0, blk * QRC), (D, QRC)).astype(BF)
    wo_j = lax.dynamic_slice(Wo, (blk * KC, 0), (KC, D)).astype(BF)
    q, qr, kr, kj, vj = _kvq(xb, Wdkv.astype(BF), wukb, wuvb,
                             wq_j, wqr_j, Wkr.astype(BF))

    oj = _attention(q, kj, vj, qr, kr)
    out16 = _outar(oj, wo_j)
    return out16.reshape(M, D).astype(jnp.float32).reshape(B, S, D)


# device time: 200943 ns/iter; 2.2011x vs baseline; 1.0110x over previous
import jax
import jax.numpy as jnp
from jax import lax
from jax.experimental import pallas as pl
from jax.experimental.pallas import tpu as pltpu

N = 8
B, S, D = 4, 256, 4096
H, Dh, Dr = 32, 128, 64
HL = H // N
KC = HL * Dh
QRC = HL * Dr
M = B * S
RC = M // N
SCALE = (Dh + Dr) ** -0.5
BF = jnp.bfloat16
VMEM = pltpu.VMEM


def _kvq_body(x_ref, wdkv_ref, wukb_ref, wuvb_ref, wq_ref, wqr_ref, wkr_ref,
              q_ref, qr_ref, kr_ref, kj_ref, vj_ref,
              c_s, call_s, wka_s, wva_s, cs, cr, wks, wkrv, wvs, wvrv):
    i = lax.axis_index("i")
    blk = (i + 1) % N

    c_s[...] = jnp.dot(x_ref[...], wdkv_ref[...],
                       preferred_element_type=jnp.float32).astype(BF)

    bsem = pltpu.get_barrier_semaphore()
    for d in range(1, N):
        pl.semaphore_signal(bsem, inc=1, device_id=((i + d) % N,),
                            device_id_type=pl.DeviceIdType.MESH)
    pl.semaphore_wait(bsem, N - 1)

    descs = []
    for d in range(1, N):
        tgt = (i + d) % N
        blk_tgt = (tgt + 1) % N
        for src, dst, ssem, rsem in (
            (c_s, call_s, cs, cr),
            (wukb_ref.at[blk_tgt], wka_s, wks, wkrv),
            (wuvb_ref.at[blk_tgt], wva_s, wvs, wvrv),
        ):
            rdma = pltpu.make_async_remote_copy(
                src_ref=src,
                dst_ref=dst.at[d - 1],
                send_sem=ssem.at[d - 1],
                recv_sem=rsem.at[d - 1],
                device_id=(tgt,),
                device_id_type=pl.DeviceIdType.MESH,
            )
            rdma.start()
            descs.append(rdma)

    call_s[N - 1] = c_s[...]
    wka_s[N - 1] = wukb_ref[blk]
    wva_s[N - 1] = wuvb_ref[blk]
    x = x_ref[...]
    q_ref[...] = jnp.dot(x, wq_ref[...],
                         preferred_element_type=jnp.float32).astype(BF)
    qr_ref[...] = jnp.dot(x, wqr_ref[...],
                          preferred_element_type=jnp.float32).astype(BF)
    kr_ref[...] = jnp.dot(x, wkr_ref[...],
                          preferred_element_type=jnp.float32).astype(BF)

    for rdma in descs:
        rdma.wait()

    k = jnp.zeros((M, KC), jnp.float32)
    v = jnp.zeros((M, KC), jnp.float32)
    for g in range(N):
        cg = call_s[g]
        k = k + jnp.dot(cg, wka_s[g], preferred_element_type=jnp.float32)
        v = v + jnp.dot(cg, wva_s[g], preferred_element_type=jnp.float32)
    kj_ref[...] = k.astype(BF)
    vj_ref[...] = v.astype(BF)

    def _exit(sem):
        for d in range(1, N):
            pl.semaphore_signal(sem, inc=1, device_id=((i + d) % N,),
                                device_id_type=pl.DeviceIdType.MESH)
        pl.semaphore_wait(sem, N - 1)

    pl.run_scoped(_exit, pltpu.SemaphoreType.REGULAR)


def _kvq(xb, wdkv, wukb, wuvb, wq_j, wqr_j, wkr):
    dc = wdkv.shape[1]
    return pl.pallas_call(
        _kvq_body,
        out_shape=[jax.ShapeDtypeStruct((M, KC), BF),
                   jax.ShapeDtypeStruct((M, QRC), BF),
                   jax.ShapeDtypeStruct((M, Dr), BF),
                   jax.ShapeDtypeStruct((M, KC), BF),
                   jax.ShapeDtypeStruct((M, KC), BF)],
        in_specs=[pl.BlockSpec(memory_space=VMEM)] * 7,
        out_specs=[pl.BlockSpec(memory_space=VMEM)] * 5,
        scratch_shapes=[
            VMEM((M, dc), BF),
            VMEM((N, M, dc), BF),
            VMEM((N, dc, KC), BF),
            VMEM((N, dc, KC), BF),
            pltpu.SemaphoreType.DMA((N - 1,)),
            pltpu.SemaphoreType.DMA((N - 1,)),
            pltpu.SemaphoreType.DMA((N - 1,)),
            pltpu.SemaphoreType.DMA((N - 1,)),
            pltpu.SemaphoreType.DMA((N - 1,)),
            pltpu.SemaphoreType.DMA((N - 1,)),
        ],
        compiler_params=pltpu.CompilerParams(collective_id=0),
    )(xb, wdkv, wukb, wuvb, wq_j, wqr_j, wkr)


def _attn_body(q_ref, k_ref, v_ref, qr_ref, kr_ref, o_ref):
    kr = kr_ref[...]
    nums = (((1,), (1,)), ((), ()))
    for h in range(HL):
        hd = slice(h * Dh, (h + 1) * Dh)
        hr = slice(h * Dr, (h + 1) * Dr)
        s = (lax.dot_general(q_ref[:, hd], k_ref[:, hd], nums,
                             preferred_element_type=jnp.float32)
             + lax.dot_general(qr_ref[:, hr], kr, nums,
                               preferred_element_type=jnp.float32)) * SCALE
        m = jnp.max(s, axis=-1, keepdims=True)
        e = jnp.exp(s - m)
        p = (e / jnp.sum(e, axis=-1, keepdims=True)).astype(BF)
        o_ref[:, hd] = jnp.dot(p, v_ref[:, hd],
                               preferred_element_type=jnp.float32).astype(BF)


def _attention(q, kj, vj, qr, kr):
    return pl.pallas_call(
        _attn_body,
        grid=(B,),
        out_shape=jax.ShapeDtypeStruct((M, KC), BF),
        in_specs=[
            pl.BlockSpec((S, KC), lambda b: (b, 0)),
            pl.BlockSpec((S, KC), lambda b: (b, 0)),
            pl.BlockSpec((S, KC), lambda b: (b, 0)),
            pl.BlockSpec((S, QRC), lambda b: (b, 0)),
            pl.BlockSpec((S, Dr), lambda b: (b, 0)),
        ],
        out_specs=pl.BlockSpec((S, KC), lambda b: (b, 0)),
    )(q, kj, vj, qr, kr)


RH = RC // 2


def _outar_body(q_ref, kj_ref, vj_ref, qr_ref, kr_ref, wo_ref, out_ref,
                buf_r, buf_l,
                rs_r_s, rs_r_r, rs_l_s, rs_l_r,
                ag_r_s, ag_r_r, ag_l_s, ag_l_r):
    i = lax.axis_index("i")
    left = (i - 1) % N
    right = (i + 1) % N
    bsem = pltpu.get_barrier_semaphore()
    for nbr in (left, right):
        pl.semaphore_signal(bsem, inc=1, device_id=(nbr,),
                            device_id_type=pl.DeviceIdType.MESH)
    pl.semaphore_wait(bsem, 2)

    wo = wo_ref[...]
    nums = (((1,), (1,)), ((), ()))

    def part(sub):
        b0 = (sub // (S // RH)) * S
        qs = q_ref[pl.ds(sub * RH, RH), :]
        qrs = qr_ref[pl.ds(sub * RH, RH), :]
        kb = kj_ref[pl.ds(b0, S), :]
        vb = vj_ref[pl.ds(b0, S), :]
        krb = kr_ref[pl.ds(b0, S), :]
        ohs = []
        for h in range(HL):
            hd = slice(h * Dh, (h + 1) * Dh)
            hr = slice(h * Dr, (h + 1) * Dr)
            s = (lax.dot_general(qs[:, hd], kb[:, hd], nums,
                                 preferred_element_type=jnp.float32)
                 + lax.dot_general(qrs[:, hr], krb, nums,
                                   preferred_element_type=jnp.float32)) * SCALE
            m = jnp.max(s, axis=-1, keepdims=True)
            e = jnp.exp(s - m)
            p = (e / jnp.sum(e, axis=-1, keepdims=True)).astype(BF)
            ohs.append(jnp.dot(p, vb[:, hd],
                               preferred_element_type=jnp.float32).astype(BF))
        o_rows = jnp.concatenate(ohs, axis=1)
        return jnp.dot(o_rows, wo, preferred_element_type=jnp.float32)

    pending = []
    buf_r[N - 1] = part(2 * i).astype(BF)
    buf_l[N - 1] = part(2 * i + 1).astype(BF)
    for s in range(N - 1):
        src = N - 1 if s == 0 else s - 1
        rd_r = pltpu.make_async_remote_copy(
            src_ref=buf_r.at[src], dst_ref=buf_r.at[s],
            send_sem=rs_r_s.at[s], recv_sem=rs_r_r.at[s],
            device_id=(right,), device_id_type=pl.DeviceIdType.MESH)
        rd_l = pltpu.make_async_remote_copy(
            src_ref=buf_l.at[src], dst_ref=buf_l.at[s],
            send_sem=rs_l_s.at[s], recv_sem=rs_l_r.at[s],
            device_id=(left,), device_id_type=pl.DeviceIdType.MESH)
        rd_r.start()
        rd_l.start()
        pending += [rd_r, rd_l]
        nxt_r = part(2 * ((i - s - 1) % N))
        nxt_l = part(2 * ((i + s + 1) % N) + 1)
        rd_r.wait_recv()
        rd_l.wait_recv()
        if s < N - 2:
            buf_r[s] = (buf_r[s] + nxt_r).astype(BF)
            buf_l[s] = (buf_l[s] + nxt_l).astype(BF)
        else:
            out_ref[2 * ((i + 1) % N)] = (buf_r[s] + nxt_r).astype(BF)
            out_ref[2 * ((i - 1) % N) + 1] = (buf_l[s] + nxt_l).astype(BF)

    for s in range(N - 1):
        sub_r = 2 * ((i + 1 - s) % N)
        sub_l = 2 * ((i - 1 + s) % N) + 1
        rd_r = pltpu.make_async_remote_copy(
            src_ref=out_ref.at[sub_r], dst_ref=out_ref.at[sub_r],
            send_sem=ag_r_s.at[s], recv_sem=ag_r_r.at[s],
            device_id=(right,), device_id_type=pl.DeviceIdType.MESH)
        rd_l = pltpu.make_async_remote_copy(
            src_ref=out_ref.at[sub_l], dst_ref=out_ref.at[sub_l],
            send_sem=ag_l_s.at[s], recv_sem=ag_l_r.at[s],
            device_id=(left,), device_id_type=pl.DeviceIdType.MESH)
        rd_r.start()
        rd_l.start()
        pending += [rd_r, rd_l]
        rd_r.wait_recv()
        rd_l.wait_recv()

    for rd in pending:
        rd.wait_send()


def _outar(q, kj, vj, qr, kr, wo_j):
    return pl.pallas_call(
        _outar_body,
        out_shape=jax.ShapeDtypeStruct((2 * N, RH, D), BF),
        in_specs=[pl.BlockSpec(memory_space=VMEM)] * 6,
        out_specs=pl.BlockSpec(memory_space=VMEM),
        scratch_shapes=[
            VMEM((N, RH, D), BF),
            VMEM((N, RH, D), BF),
        ] + [pltpu.SemaphoreType.DMA((N - 1,))] * 8,
        compiler_params=pltpu.CompilerParams(collective_id=1),
    )(q, kj, vj, qr, kr, wo_j)


def kernel(x, Wdkv, Wuk, Wuv, Wq, Wqr, Wkr, Wo):
    i = lax.axis_index("i")
    blk = (i + 1) % N

    xb = x.reshape(M, D).astype(BF)
    dc = Wdkv.shape[1]
    wukb = Wuk.astype(BF).reshape(dc, N, KC).transpose(1, 0, 2)
    wuvb = Wuv.astype(BF).reshape(dc, N, KC).transpose(1, 0, 2)

    wq_j = lax.dynamic_slice(Wq, (0, blk * KC), (D, KC)).astype(BF)
    wqr_j = lax.dynamic_slice(Wqr, (0, blk * QRC), (D, QRC)).astype(BF)
    wo_j = lax.dynamic_slice(Wo, (blk * KC, 0), (KC, D)).astype(BF)
    q, qr, kr, kj, vj = _kvq(xb, Wdkv.astype(BF), wukb, wuvb,
                             wq_j, wqr_j, Wkr.astype(BF))

    out16 = _outar(q, kj, vj, qr, kr, wo_j)
    return out16.reshape(M, D).reshape(B, S, D)


# device time: 181421 ns/iter; 2.4379x vs baseline; 1.1076x over previous
import jax
import jax.numpy as jnp
from jax import lax
from jax.experimental import pallas as pl
from jax.experimental.pallas import tpu as pltpu

N = 8
B, S, D = 4, 256, 4096
H, Dh, Dr = 32, 128, 64
HL = H // N
KC = HL * Dh
QRC = HL * Dr
M = B * S
RC = M // N
SCALE = (Dh + Dr) ** -0.5
BF = jnp.bfloat16
VMEM = pltpu.VMEM


def _kvq_body(x_ref, wdkv_ref, wukb_ref, wuvb_ref, wq_ref, wqr_ref, wkr_ref,
              q_ref, qr_ref, kr_ref, kj_ref, vj_ref,
              c_s, call_s, wka_s, wva_s, cs, cr, wks, wkrv, wvs, wvrv):
    i = lax.axis_index("i")
    blk = (i + 1) % N

    c_s[...] = jnp.dot(x_ref[...], wdkv_ref[...],
                       preferred_element_type=jnp.float32).astype(BF)

    bsem = pltpu.get_barrier_semaphore()
    for d in range(1, N):
        pl.semaphore_signal(bsem, inc=1, device_id=((i + d) % N,),
                            device_id_type=pl.DeviceIdType.MESH)
    pl.semaphore_wait(bsem, N - 1)

    descs = []
    for d in range(1, N):
        tgt = (i + d) % N
        blk_tgt = (tgt + 1) % N
        for src, dst, ssem, rsem in (
            (c_s, call_s, cs, cr),
            (wukb_ref.at[blk_tgt], wka_s, wks, wkrv),
            (wuvb_ref.at[blk_tgt], wva_s, wvs, wvrv),
        ):
            rdma = pltpu.make_async_remote_copy(
                src_ref=src,
                dst_ref=dst.at[d - 1],
                send_sem=ssem.at[d - 1],
                recv_sem=rsem.at[d - 1],
                device_id=(tgt,),
                device_id_type=pl.DeviceIdType.MESH,
            )
            rdma.start()
            descs.append(rdma)

    call_s[N - 1] = c_s[...]
    wka_s[N - 1] = wukb_ref[blk]
    wva_s[N - 1] = wuvb_ref[blk]
    x = x_ref[...]
    q_ref[...] = jnp.dot(x, wq_ref[...],
                         preferred_element_type=jnp.float32).astype(BF)
    qr_ref[...] = jnp.dot(x, wqr_ref[...],
                          preferred_element_type=jnp.float32).astype(BF)
    kr_ref[...] = jnp.dot(x, wkr_ref[...],
                          preferred_element_type=jnp.float32).astype(BF)

    for rdma in descs:
        rdma.wait()

    k = jnp.zeros((M, KC), jnp.float32)
    v = jnp.zeros((M, KC), jnp.float32)
    for g in range(N):
        cg = call_s[g]
        k = k + jnp.dot(cg, wka_s[g], preferred_element_type=jnp.float32)
        v = v + jnp.dot(cg, wva_s[g], preferred_element_type=jnp.float32)
    kj_ref[...] = k.astype(BF)
    vj_ref[...] = v.astype(BF)

    def _exit(sem):
        for d in range(1, N):
            pl.semaphore_signal(sem, inc=1, device_id=((i + d) % N,),
                                device_id_type=pl.DeviceIdType.MESH)
        pl.semaphore_wait(sem, N - 1)

    pl.run_scoped(_exit, pltpu.SemaphoreType.REGULAR)


def _kvq(xb, wdkv, wukb, wuvb, wq_j, wqr_j, wkr):
    dc = wdkv.shape[1]
    return pl.pallas_call(
        _kvq_body,
        out_shape=[jax.ShapeDtypeStruct((M, KC), BF),
                   jax.ShapeDtypeStruct((M, QRC), BF),
                   jax.ShapeDtypeStruct((M, Dr), BF),
                   jax.ShapeDtypeStruct((M, KC), BF),
                   jax.ShapeDtypeStruct((M, KC), BF)],
        in_specs=[pl.BlockSpec(memory_space=VMEM)] * 7,
        out_specs=[pl.BlockSpec(memory_space=VMEM)] * 5,
        scratch_shapes=[
            VMEM((M, dc), BF),
            VMEM((N, M, dc), BF),
            VMEM((N, dc, KC), BF),
            VMEM((N, dc, KC), BF),
            pltpu.SemaphoreType.DMA((N - 1,)),
            pltpu.SemaphoreType.DMA((N - 1,)),
            pltpu.SemaphoreType.DMA((N - 1,)),
            pltpu.SemaphoreType.DMA((N - 1,)),
            pltpu.SemaphoreType.DMA((N - 1,)),
            pltpu.SemaphoreType.DMA((N - 1,)),
        ],
        compiler_params=pltpu.CompilerParams(collective_id=0),
    )(xb, wdkv, wukb, wuvb, wq_j, wqr_j, wkr)


def _attn_body(q_ref, k_ref, v_ref, qr_ref, kr_ref, o_ref):
    kr = kr_ref[...]
    nums = (((1,), (1,)), ((), ()))
    for h in range(HL):
        hd = slice(h * Dh, (h + 1) * Dh)
        hr = slice(h * Dr, (h + 1) * Dr)
        s = (lax.dot_general(q_ref[:, hd], k_ref[:, hd], nums,
                             preferred_element_type=jnp.float32)
             + lax.dot_general(qr_ref[:, hr], kr, nums,
                               preferred_element_type=jnp.float32)) * SCALE
        m = jnp.max(s, axis=-1, keepdims=True)
        e = jnp.exp(s - m)
        p = (e / jnp.sum(e, axis=-1, keepdims=True)).astype(BF)
        o_ref[:, hd] = jnp.dot(p, v_ref[:, hd],
                               preferred_element_type=jnp.float32).astype(BF)


def _attention(q, kj, vj, qr, kr):
    return pl.pallas_call(
        _attn_body,
        grid=(B,),
        out_shape=jax.ShapeDtypeStruct((M, KC), BF),
        in_specs=[
            pl.BlockSpec((S, KC), lambda b: (b, 0)),
            pl.BlockSpec((S, KC), lambda b: (b, 0)),
            pl.BlockSpec((S, KC), lambda b: (b, 0)),
            pl.BlockSpec((S, QRC), lambda b: (b, 0)),
            pl.BlockSpec((S, Dr), lambda b: (b, 0)),
        ],
        out_specs=pl.BlockSpec((S, KC), lambda b: (b, 0)),
    )(q, kj, vj, qr, kr)


RH = RC // 2


def _outar_body(q_ref, kj_ref, vj_ref, qr_ref, kr_ref, wo_ref, out_ref,
                buf_r, buf_l,
                rs_r_s, rs_r_r, rs_l_s, rs_l_r,
                ag_s, ag_r):
    i = lax.axis_index("i")
    left = (i - 1) % N
    right = (i + 1) % N
    bsem = pltpu.get_barrier_semaphore()
    for nbr in (left, right):
        pl.semaphore_signal(bsem, inc=1, device_id=(nbr,),
                            device_id_type=pl.DeviceIdType.MESH)
    pl.semaphore_wait(bsem, 2)

    wo = wo_ref[...]
    nums = (((1,), (1,)), ((), ()))

    def part(sub):
        b0 = (sub // (S // RH)) * S
        qs = q_ref[pl.ds(sub * RH, RH), :]
        qrs = qr_ref[pl.ds(sub * RH, RH), :]
        kb = kj_ref[pl.ds(b0, S), :]
        vb = vj_ref[pl.ds(b0, S), :]
        krb = kr_ref[pl.ds(b0, S), :]
        ohs = []
        for h in range(HL):
            hd = slice(h * Dh, (h + 1) * Dh)
            hr = slice(h * Dr, (h + 1) * Dr)
            s = (lax.dot_general(qs[:, hd], kb[:, hd], nums,
                                 preferred_element_type=jnp.float32)
                 + lax.dot_general(qrs[:, hr], krb, nums,
                                   preferred_element_type=jnp.float32)) * SCALE
            m = jnp.max(s, axis=-1, keepdims=True)
            e = jnp.exp(s - m)
            p = (e / jnp.sum(e, axis=-1, keepdims=True)).astype(BF)
            ohs.append(jnp.dot(p, vb[:, hd],
                               preferred_element_type=jnp.float32).astype(BF))
        o_rows = jnp.concatenate(ohs, axis=1)
        return jnp.dot(o_rows, wo, preferred_element_type=jnp.float32)

    pending = []
    buf_r[N - 1] = part(2 * i).astype(BF)
    buf_l[N - 1] = part(2 * i + 1).astype(BF)
    for s in range(N - 1):
        src = N - 1 if s == 0 else s - 1
        rd_r = pltpu.make_async_remote_copy(
            src_ref=buf_r.at[src], dst_ref=buf_r.at[s],
            send_sem=rs_r_s.at[s], recv_sem=rs_r_r.at[s],
            device_id=(right,), device_id_type=pl.DeviceIdType.MESH)
        rd_l = pltpu.make_async_remote_copy(
            src_ref=buf_l.at[src], dst_ref=buf_l.at[s],
            send_sem=rs_l_s.at[s], recv_sem=rs_l_r.at[s],
            device_id=(left,), device_id_type=pl.DeviceIdType.MESH)
        rd_r.start()
        rd_l.start()
        pending += [rd_r, rd_l]
        nxt_r = part(2 * ((i - s - 1) % N))
        nxt_l = part(2 * ((i + s + 1) % N) + 1)
        rd_r.wait_recv()
        rd_l.wait_recv()
        if s < N - 2:
            buf_r[s] = (buf_r[s] + nxt_r).astype(BF)
            buf_l[s] = (buf_l[s] + nxt_l).astype(BF)
        else:
            out_ref[2 * ((i + 1) % N)] = (buf_r[s] + nxt_r).astype(BF)
            out_ref[2 * ((i - 1) % N) + 1] = (buf_l[s] + nxt_l).astype(BF)

    def PX(d):
        return d ^ 1

    def PY(d):
        return 4 * (d // 4) + (3 - d % 4)

    def PZ(d):
        return d ^ 4

    tstarts = (0, 1024, 2560)
    twidths = (1024, 1536, 1536)
    orders = ((PX, PY, PZ), (PY, PZ, PX), (PZ, PX, PY))
    held = [[i], [i], [i]]
    ctr = [0, 0, 0]
    for p in range(3):
        phase_descs = []
        for t in range(3):
            P = orders[t][p]
            prt = P(i)
            for d in held[t]:
                for sub in (2 * ((d + 1) % N), 2 * ((d - 1) % N) + 1):
                    piece = out_ref.at[sub, :, pl.ds(tstarts[t], twidths[t])]
                    rd = pltpu.make_async_remote_copy(
                        src_ref=piece, dst_ref=piece,
                        send_sem=ag_s.at[t, ctr[t]],
                        recv_sem=ag_r.at[t, ctr[t]],
                        device_id=(prt,),
                        device_id_type=pl.DeviceIdType.MESH)
                    rd.start()
                    ctr[t] += 1
                    phase_descs.append(rd)
                    pending.append(rd)
            held[t] = held[t] + [P(d) for d in held[t]]
        for rd in phase_descs:
            rd.wait_recv()

    for rd in pending:
        rd.wait_send()


def _outar(q, kj, vj, qr, kr, wo_j):
    return pl.pallas_call(
        _outar_body,
        out_shape=jax.ShapeDtypeStruct((2 * N, RH, D), BF),
        in_specs=[pl.BlockSpec(memory_space=VMEM)] * 6,
        out_specs=pl.BlockSpec(memory_space=VMEM),
        scratch_shapes=[
            VMEM((N, RH, D), BF),
            VMEM((N, RH, D), BF),
            pltpu.SemaphoreType.DMA((N - 1,)),
            pltpu.SemaphoreType.DMA((N - 1,)),
            pltpu.SemaphoreType.DMA((N - 1,)),
            pltpu.SemaphoreType.DMA((N - 1,)),
            pltpu.SemaphoreType.DMA((3, 14)),
            pltpu.SemaphoreType.DMA((3, 14)),
        ],
        compiler_params=pltpu.CompilerParams(collective_id=1),
    )(q, kj, vj, qr, kr, wo_j)


def kernel(x, Wdkv, Wuk, Wuv, Wq, Wqr, Wkr, Wo):
    i = lax.axis_index("i")
    blk = (i + 1) % N

    xb = x.reshape(M, D).astype(BF)
    dc = Wdkv.shape[1]
    wukb = Wuk.astype(BF).reshape(dc, N, KC).transpose(1, 0, 2)
    wuvb = Wuv.astype(BF).reshape(dc, N, KC).transpose(1, 0, 2)

    wq_j = lax.dynamic_slice(Wq, (0, blk * KC), (D, KC)).astype(BF)
    wqr_j = lax.dynamic_slice(Wqr, (0, blk * QRC), (D, QRC)).astype(BF)
    wo_j = lax.dynamic_slice(Wo, (blk * KC, 0), (KC, D)).astype(BF)
    q, qr, kr, kj, vj = _kvq(xb, Wdkv.astype(BF), wukb, wuvb,
                             wq_j, wqr_j, Wkr.astype(BF))

    out16 = _outar(q, kj, vj, qr, kr, wo_j)
    return out16.reshape(M, D).reshape(B, S, D)


# device time: 172747 ns/iter; 2.5603x vs baseline; 1.0502x over previous
import jax
import jax.numpy as jnp
from jax import lax
from jax.experimental import pallas as pl
from jax.experimental.pallas import tpu as pltpu

N = 8
B, S, D = 4, 256, 4096
H, Dh, Dr = 32, 128, 64
HL = H // N
KC = HL * Dh
QRC = HL * Dr
M = B * S
RC = M // N
SCALE = (Dh + Dr) ** -0.5
BF = jnp.bfloat16
VMEM = pltpu.VMEM


def _kvq_body(x_ref, wdkv_ref, wukb_ref, wuvb_ref, wq_ref, wqr_ref, wkr_ref,
              q_ref, qr_ref, kr_ref, kj_ref, vj_ref,
              c_s, call_s, wka_s, wva_s, cs, cr, wks, wkrv, wvs, wvrv):
    i = lax.axis_index("i")
    blk = (i + 1) % N

    c_s[...] = jnp.dot(x_ref[...], wdkv_ref[...],
                       preferred_element_type=jnp.float32).astype(BF)

    bsem = pltpu.get_barrier_semaphore()
    for d in range(1, N):
        pl.semaphore_signal(bsem, inc=1, device_id=((i + d) % N,),
                            device_id_type=pl.DeviceIdType.MESH)
    pl.semaphore_wait(bsem, N - 1)

    dc = c_s.shape[1]
    descs = []
    for d in range(1, N):
        tgt = (i + d) % N
        blk_tgt = (tgt + 1) % N
        for src, dst, ssem, rsem in (
            (c_s, call_s.at[:, pl.ds((d - 1) * dc, dc)], cs, cr),
            (wukb_ref.at[blk_tgt],
             wka_s.at[pl.ds((d - 1) * dc, dc), :], wks, wkrv),
            (wuvb_ref.at[blk_tgt],
             wva_s.at[pl.ds((d - 1) * dc, dc), :], wvs, wvrv),
        ):
            rdma = pltpu.make_async_remote_copy(
                src_ref=src,
                dst_ref=dst,
                send_sem=ssem.at[d - 1],
                recv_sem=rsem.at[d - 1],
                device_id=(tgt,),
                device_id_type=pl.DeviceIdType.MESH,
            )
            rdma.start()
            descs.append(rdma)

    call_s[:, (N - 1) * dc:] = c_s[...]
    wka_s[(N - 1) * dc:, :] = wukb_ref[blk]
    wva_s[(N - 1) * dc:, :] = wuvb_ref[blk]
    x = x_ref[...]
    q_ref[...] = jnp.dot(x, wq_ref[...],
                         preferred_element_type=jnp.float32).astype(BF)
    qr_ref[...] = jnp.dot(x, wqr_ref[...],
                          preferred_element_type=jnp.float32).astype(BF)
    kr_ref[...] = jnp.dot(x, wkr_ref[...],
                          preferred_element_type=jnp.float32).astype(BF)

    for rdma in descs:
        rdma.wait()

    cfull = call_s[...]
    kj_ref[...] = jnp.dot(cfull, wka_s[...],
                          preferred_element_type=jnp.float32).astype(BF)
    vj_ref[...] = jnp.dot(cfull, wva_s[...],
                          preferred_element_type=jnp.float32).astype(BF)

    def _exit(sem):
        for d in range(1, N):
            pl.semaphore_signal(sem, inc=1, device_id=((i + d) % N,),
                                device_id_type=pl.DeviceIdType.MESH)
        pl.semaphore_wait(sem, N - 1)

    pl.run_scoped(_exit, pltpu.SemaphoreType.REGULAR)


def _kvq(xb, wdkv, wukb, wuvb, wq_j, wqr_j, wkr):
    dc = wdkv.shape[1]
    return pl.pallas_call(
        _kvq_body,
        out_shape=[jax.ShapeDtypeStruct((M, KC), BF),
                   jax.ShapeDtypeStruct((M, QRC), BF),
                   jax.ShapeDtypeStruct((M, Dr), BF),
                   jax.ShapeDtypeStruct((M, KC), BF),
                   jax.ShapeDtypeStruct((M, KC), BF)],
        in_specs=[pl.BlockSpec(memory_space=VMEM)] * 7,
        out_specs=[pl.BlockSpec(memory_space=VMEM)] * 5,
        scratch_shapes=[
            VMEM((M, dc), BF),
            VMEM((M, N * dc), BF),
            VMEM((N * dc, KC), BF),
            VMEM((N * dc, KC), BF),
            pltpu.SemaphoreType.DMA((N - 1,)),
            pltpu.SemaphoreType.DMA((N - 1,)),
            pltpu.SemaphoreType.DMA((N - 1,)),
            pltpu.SemaphoreType.DMA((N - 1,)),
            pltpu.SemaphoreType.DMA((N - 1,)),
            pltpu.SemaphoreType.DMA((N - 1,)),
        ],
        compiler_params=pltpu.CompilerParams(collective_id=0),
    )(xb, wdkv, wukb, wuvb, wq_j, wqr_j, wkr)


def _attn_body(q_ref, k_ref, v_ref, qr_ref, kr_ref, o_ref):
    kr = kr_ref[...]
    nums = (((1,), (1,)), ((), ()))
    for h in range(HL):
        hd = slice(h * Dh, (h + 1) * Dh)
        hr = slice(h * Dr, (h + 1) * Dr)
        s = (lax.dot_general(q_ref[:, hd], k_ref[:, hd], nums,
                             preferred_element_type=jnp.float32)
             + lax.dot_general(qr_ref[:, hr], kr, nums,
                               preferred_element_type=jnp.float32)) * SCALE
        m = jnp.max(s, axis=-1, keepdims=True)
        e = jnp.exp(s - m)
        p = (e / jnp.sum(e, axis=-1, keepdims=True)).astype(BF)
        o_ref[:, hd] = jnp.dot(p, v_ref[:, hd],
                               preferred_element_type=jnp.float32).astype(BF)


def _attention(q, kj, vj, qr, kr):
    return pl.pallas_call(
        _attn_body,
        grid=(B,),
        out_shape=jax.ShapeDtypeStruct((M, KC), BF),
        in_specs=[
            pl.BlockSpec((S, KC), lambda b: (b, 0)),
            pl.BlockSpec((S, KC), lambda b: (b, 0)),
            pl.BlockSpec((S, KC), lambda b: (b, 0)),
            pl.BlockSpec((S, QRC), lambda b: (b, 0)),
            pl.BlockSpec((S, Dr), lambda b: (b, 0)),
        ],
        out_specs=pl.BlockSpec((S, KC), lambda b: (b, 0)),
    )(q, kj, vj, qr, kr)


RH = RC // 2


def _outar_body(q_ref, kj_ref, vj_ref, qr_ref, kr_ref, wo_ref, out_ref,
                buf_r, buf_l,
                rs_r_s, rs_r_r, rs_l_s, rs_l_r,
                ag_s, ag_r):
    i = lax.axis_index("i")
    left = (i - 1) % N
    right = (i + 1) % N
    bsem = pltpu.get_barrier_semaphore()
    for nbr in (left, right):
        pl.semaphore_signal(bsem, inc=1, device_id=(nbr,),
                            device_id_type=pl.DeviceIdType.MESH)
    pl.semaphore_wait(bsem, 2)

    wo = wo_ref[...]
    nums = (((1,), (1,)), ((), ()))

    def part(sub):
        b0 = (sub // (S // RH)) * S
        qs = q_ref[pl.ds(sub * RH, RH), :]
        qrs = qr_ref[pl.ds(sub * RH, RH), :]
        kb = kj_ref[pl.ds(b0, S), :]
        vb = vj_ref[pl.ds(b0, S), :]
        krb = kr_ref[pl.ds(b0, S), :]
        ohs = []
        for h in range(HL):
            hd = slice(h * Dh, (h + 1) * Dh)
            hr = slice(h * Dr, (h + 1) * Dr)
            s = (lax.dot_general(qs[:, hd], kb[:, hd], nums,
                                 preferred_element_type=jnp.float32)
                 + lax.dot_general(qrs[:, hr], krb, nums,
                                   preferred_element_type=jnp.float32)) * SCALE
            m = jnp.max(s, axis=-1, keepdims=True)
            e = jnp.exp(s - m)
            p = (e / jnp.sum(e, axis=-1, keepdims=True)).astype(BF)
            ohs.append(jnp.dot(p, vb[:, hd],
                               preferred_element_type=jnp.float32).astype(BF))
        o_rows = jnp.concatenate(ohs, axis=1)
        return jnp.dot(o_rows, wo, preferred_element_type=jnp.float32)

    pending = []
    buf_r[N - 1] = part(2 * i).astype(BF)
    buf_l[N - 1] = part(2 * i + 1).astype(BF)
    for s in range(N - 1):
        src = N - 1 if s == 0 else s - 1
        rd_r = pltpu.make_async_remote_copy(
            src_ref=buf_r.at[src], dst_ref=buf_r.at[s],
            send_sem=rs_r_s.at[s], recv_sem=rs_r_r.at[s],
            device_id=(right,), device_id_type=pl.DeviceIdType.MESH)
        rd_l = pltpu.make_async_remote_copy(
            src_ref=buf_l.at[src], dst_ref=buf_l.at[s],
            send_sem=rs_l_s.at[s], recv_sem=rs_l_r.at[s],
            device_id=(left,), device_id_type=pl.DeviceIdType.MESH)
        rd_r.start()
        rd_l.start()
        pending += [rd_r, rd_l]
        nxt_r = part(2 * ((i - s - 1) % N))
        nxt_l = part(2 * ((i + s + 1) % N) + 1)
        rd_r.wait_recv()
        rd_l.wait_recv()
        if s < N - 2:
            buf_r[s] = (buf_r[s] + nxt_r).astype(BF)
            buf_l[s] = (buf_l[s] + nxt_l).astype(BF)
        else:
            out_ref[2 * ((i + 1) % N)] = (buf_r[s] + nxt_r).astype(BF)
            out_ref[2 * ((i - 1) % N) + 1] = (buf_l[s] + nxt_l).astype(BF)

    def PX(d):
        return d ^ 1

    def PY(d):
        return 4 * (d // 4) + (3 - d % 4)

    def PZ(d):
        return d ^ 4

    tstarts = (0, 1024, 2560)
    twidths = (1024, 1536, 1536)
    orders = ((PX, PY, PZ), (PY, PZ, PX), (PZ, PX, PY))
    held = [[i], [i], [i]]
    ctr = [0, 0, 0]
    for p in range(3):
        phase_descs = []
        for t in range(3):
            P = orders[t][p]
            prt = P(i)
            for d in held[t]:
                for sub in (2 * ((d + 1) % N), 2 * ((d - 1) % N) + 1):
                    piece = out_ref.at[sub, :, pl.ds(tstarts[t], twidths[t])]
                    rd = pltpu.make_async_remote_copy(
                        src_ref=piece, dst_ref=piece,
                        send_sem=ag_s.at[t, ctr[t]],
                        recv_sem=ag_r.at[t, ctr[t]],
                        device_id=(prt,),
                        device_id_type=pl.DeviceIdType.MESH)
                    rd.start()
                    ctr[t] += 1
                    phase_descs.append(rd)
                    pending.append(rd)
            held[t] = held[t] + [P(d) for d in held[t]]
        for rd in phase_descs:
            rd.wait_recv()

    for rd in pending:
        rd.wait_send()


def _outar(q, kj, vj, qr, kr, wo_j):
    return pl.pallas_call(
        _outar_body,
        out_shape=jax.ShapeDtypeStruct((2 * N, RH, D), BF),
        in_specs=[pl.BlockSpec(memory_space=VMEM)] * 6,
        out_specs=pl.BlockSpec(memory_space=VMEM),
        scratch_shapes=[
            VMEM((N, RH, D), BF),
            VMEM((N, RH, D), BF),
            pltpu.SemaphoreType.DMA((N - 1,)),
            pltpu.SemaphoreType.DMA((N - 1,)),
            pltpu.SemaphoreType.DMA((N - 1,)),
            pltpu.SemaphoreType.DMA((N - 1,)),
            pltpu.SemaphoreType.DMA((3, 14)),
            pltpu.SemaphoreType.DMA((3, 14)),
        ],
        compiler_params=pltpu.CompilerParams(collective_id=1),
    )(q, kj, vj, qr, kr, wo_j)


def kernel(x, Wdkv, Wuk, Wuv, Wq, Wqr, Wkr, Wo):
    i = lax.axis_index("i")
    blk = (i + 1) % N

    xb = x.reshape(M, D).astype(BF)
    dc = Wdkv.shape[1]
    wukb = Wuk.astype(BF).reshape(dc, N, KC).transpose(1, 0, 2)
    wuvb = Wuv.astype(BF).reshape(dc, N, KC).transpose(1, 0, 2)

    wq_j = lax.dynamic_slice(Wq, (0, blk * KC), (D, KC)).astype(BF)
    wqr_j = lax.dynamic_slice(Wqr, (0, blk * QRC), (D, QRC)).astype(BF)
    wo_j = lax.dynamic_slice(Wo, (blk * KC, 0), (KC, D)).astype(BF)
    q, qr, kr, kj, vj = _kvq(xb, Wdkv.astype(BF), wukb, wuvb,
                             wq_j, wqr_j, Wkr.astype(BF))

    out16 = _outar(q, kj, vj, qr, kr, wo_j)
    return out16.reshape(M, D).reshape(B, S, D)


# device time: 162830 ns/iter; 2.7162x vs baseline; 1.0609x over previous
import jax
import jax.numpy as jnp
from jax import lax
from jax.experimental import pallas as pl
from jax.experimental.pallas import tpu as pltpu

N = 8
B, S, D = 4, 256, 4096
H, Dh, Dr = 32, 128, 64
HL = H // N
KC = HL * Dh
QRC = HL * Dr
M = B * S
RC = M // N
SCALE = (Dh + Dr) ** -0.5
BF = jnp.bfloat16
VMEM = pltpu.VMEM


def _kvq_body(x_ref, wdkv_ref, wukb_ref, wuvb_ref, wq_ref, wqr_ref, wkr_ref,
              q_ref, qr_ref, kr_ref, kj_ref, vj_ref,
              c_s, call_s, wka_s, wva_s, cs, cr, wks, wkrv, wvs, wvrv):
    i = lax.axis_index("i")
    blk = (i + 1) % N

    c_s[...] = jnp.dot(x_ref[...], wdkv_ref[...],
                       preferred_element_type=jnp.float32).astype(BF)

    bsem = pltpu.get_barrier_semaphore()
    for d in range(1, N):
        pl.semaphore_signal(bsem, inc=1, device_id=((i + d) % N,),
                            device_id_type=pl.DeviceIdType.MESH)
    pl.semaphore_wait(bsem, N - 1)

    dc = c_s.shape[1]
    descs = []
    for d in range(1, N):
        tgt = (i + d) % N
        blk_tgt = (tgt + 1) % N
        for src, dst, ssem, rsem in (
            (c_s, call_s.at[:, pl.ds((d - 1) * dc, dc)], cs, cr),
            (wukb_ref.at[blk_tgt],
             wka_s.at[pl.ds((d - 1) * dc, dc), :], wks, wkrv),
            (wuvb_ref.at[blk_tgt],
             wva_s.at[pl.ds((d - 1) * dc, dc), :], wvs, wvrv),
        ):
            rdma = pltpu.make_async_remote_copy(
                src_ref=src,
                dst_ref=dst,
                send_sem=ssem.at[d - 1],
                recv_sem=rsem.at[d - 1],
                device_id=(tgt,),
                device_id_type=pl.DeviceIdType.MESH,
            )
            rdma.start()
            descs.append(rdma)

    call_s[:, (N - 1) * dc:] = c_s[...]
    wka_s[(N - 1) * dc:, :] = wukb_ref[blk]
    wva_s[(N - 1) * dc:, :] = wuvb_ref[blk]
    x = x_ref[...]
    q_ref[...] = jnp.dot(x, wq_ref[...],
                         preferred_element_type=jnp.float32).astype(BF)
    qr_ref[...] = jnp.dot(x, wqr_ref[...],
                          preferred_element_type=jnp.float32).astype(BF)
    kr_ref[...] = jnp.dot(x, wkr_ref[...],
                          preferred_element_type=jnp.float32).astype(BF)

    for rdma in descs:
        rdma.wait()

    cfull = call_s[...]
    kj_ref[...] = jnp.dot(cfull, wka_s[...],
                          preferred_element_type=jnp.float32).astype(BF)
    vj_ref[...] = jnp.dot(cfull, wva_s[...],
                          preferred_element_type=jnp.float32).astype(BF)

    def _exit(sem):
        for d in range(1, N):
            pl.semaphore_signal(sem, inc=1, device_id=((i + d) % N,),
                                device_id_type=pl.DeviceIdType.MESH)
        pl.semaphore_wait(sem, N - 1)

    pl.run_scoped(_exit, pltpu.SemaphoreType.REGULAR)


def _kvq(xb, wdkv, wukb, wuvb, wq_j, wqr_j, wkr):
    dc = wdkv.shape[1]
    return pl.pallas_call(
        _kvq_body,
        out_shape=[jax.ShapeDtypeStruct((M, KC), BF),
                   jax.ShapeDtypeStruct((M, QRC), BF),
                   jax.ShapeDtypeStruct((M, Dr), BF),
                   jax.ShapeDtypeStruct((M, KC), BF),
                   jax.ShapeDtypeStruct((M, KC), BF)],
        in_specs=[pl.BlockSpec(memory_space=VMEM)] * 7,
        out_specs=[pl.BlockSpec(memory_space=VMEM)] * 5,
        scratch_shapes=[
            VMEM((M, dc), BF),
            VMEM((M, N * dc), BF),
            VMEM((N * dc, KC), BF),
            VMEM((N * dc, KC), BF),
            pltpu.SemaphoreType.DMA((N - 1,)),
            pltpu.SemaphoreType.DMA((N - 1,)),
            pltpu.SemaphoreType.DMA((N - 1,)),
            pltpu.SemaphoreType.DMA((N - 1,)),
            pltpu.SemaphoreType.DMA((N - 1,)),
            pltpu.SemaphoreType.DMA((N - 1,)),
        ],
        compiler_params=pltpu.CompilerParams(collective_id=0),
    )(xb, wdkv, wukb, wuvb, wq_j, wqr_j, wkr)


def _attn_body(q_ref, k_ref, v_ref, qr_ref, kr_ref, o_ref):
    kr = kr_ref[...]
    nums = (((1,), (1,)), ((), ()))
    for h in range(HL):
        hd = slice(h * Dh, (h + 1) * Dh)
        hr = slice(h * Dr, (h + 1) * Dr)
        s = (lax.dot_general(q_ref[:, hd], k_ref[:, hd], nums,
                             preferred_element_type=jnp.float32)
             + lax.dot_general(qr_ref[:, hr], kr, nums,
                               preferred_element_type=jnp.float32)) * SCALE
        m = jnp.max(s, axis=-1, keepdims=True)
        e = jnp.exp(s - m)
        p = (e / jnp.sum(e, axis=-1, keepdims=True)).astype(BF)
        o_ref[:, hd] = jnp.dot(p, v_ref[:, hd],
                               preferred_element_type=jnp.float32).astype(BF)


def _attention(q, kj, vj, qr, kr):
    return pl.pallas_call(
        _attn_body,
        grid=(B,),
        out_shape=jax.ShapeDtypeStruct((M, KC), BF),
        in_specs=[
            pl.BlockSpec((S, KC), lambda b: (b, 0)),
            pl.BlockSpec((S, KC), lambda b: (b, 0)),
            pl.BlockSpec((S, KC), lambda b: (b, 0)),
            pl.BlockSpec((S, QRC), lambda b: (b, 0)),
            pl.BlockSpec((S, Dr), lambda b: (b, 0)),
        ],
        out_specs=pl.BlockSpec((S, KC), lambda b: (b, 0)),
    )(q, kj, vj, qr, kr)


RH = RC // 2


def _outar_body(q_ref, kj_ref, vj_ref, qr_ref, kr_ref, wo_ref, out_ref,
                buf_r, buf_l,
                rs_r_s, rs_r_r, rs_l_s, rs_l_r,
                ag_s, ag_r):
    i = lax.axis_index("i")
    left = (i - 1) % N
    right = (i + 1) % N
    bsem = pltpu.get_barrier_semaphore()
    for nbr in (left, right):
        pl.semaphore_signal(bsem, inc=1, device_id=(nbr,),
                            device_id_type=pl.DeviceIdType.MESH)
    pl.semaphore_wait(bsem, 2)

    wo = wo_ref[...]
    nums = (((1,), (1,)), ((), ()))

    def part(sub):
        b0 = (sub // (S // RH)) * S
        qs = q_ref[pl.ds(sub * RH, RH), :]
        qrs = qr_ref[pl.ds(sub * RH, RH), :]
        kb = kj_ref[pl.ds(b0, S), :]
        vb = vj_ref[pl.ds(b0, S), :]
        krb = kr_ref[pl.ds(b0, S), :]
        ohs = []
        for h in range(HL):
            hd = slice(h * Dh, (h + 1) * Dh)
            hr = slice(h * Dr, (h + 1) * Dr)
            s = (lax.dot_general(qs[:, hd], kb[:, hd], nums,
                                 preferred_element_type=jnp.float32)
                 + lax.dot_general(qrs[:, hr], krb, nums,
                                   preferred_element_type=jnp.float32)) * SCALE
            m = jnp.max(s, axis=-1, keepdims=True)
            e = jnp.exp(s - m)
            p = (e / jnp.sum(e, axis=-1, keepdims=True)).astype(BF)
            ohs.append(jnp.dot(p, vb[:, hd],
                               preferred_element_type=jnp.float32).astype(BF))
        o_rows = jnp.concatenate(ohs, axis=1)
        return jnp.dot(o_rows, wo, preferred_element_type=jnp.float32)

    pending = []
    CH = D // 2
    sub_t = 2 * ((i + 1) % N)
    sub_b = 2 * ((i - 1) % N) + 1

    def rs_start(buf, ch, hop, dev, ssem, rsem):
        src = N - 1 if hop == 0 else hop - 1
        sl = pl.ds(ch * CH, CH)
        rd = pltpu.make_async_remote_copy(
            src_ref=buf.at[src, :, sl], dst_ref=buf.at[hop, :, sl],
            send_sem=ssem.at[ch, hop], recv_sem=rsem.at[ch, hop],
            device_id=(dev,), device_id_type=pl.DeviceIdType.MESH)
        rd.start()
        pending.append(rd)
        return rd

    buf_r[N - 1] = part(2 * i).astype(BF)
    buf_l[N - 1] = part(2 * i + 1).astype(BF)
    rx = rs_start(buf_r, 0, 0, right, rs_r_s, rs_r_r)
    lx = rs_start(buf_l, 0, 0, left, rs_l_s, rs_l_r)
    ry = rs_start(buf_r, 1, 0, right, rs_r_s, rs_r_r)
    ly = rs_start(buf_l, 1, 0, left, rs_l_s, rs_l_r)
    nxt_r = part(2 * ((i - 1) % N))
    nxt_l = part(2 * ((i + 1) % N) + 1)
    for s in range(N - 1):
        rx.wait_recv()
        lx.wait_recv()
        if s < N - 2:
            buf_r[s, :, :CH] = (buf_r[s, :, :CH] + nxt_r[:, :CH]).astype(BF)
            buf_l[s, :, :CH] = (buf_l[s, :, :CH] + nxt_l[:, :CH]).astype(BF)
            rx = rs_start(buf_r, 0, s + 1, right, rs_r_s, rs_r_r)
            lx = rs_start(buf_l, 0, s + 1, left, rs_l_s, rs_l_r)
            nnr = part(2 * ((i - s - 2) % N))
            nnl = part(2 * ((i + s + 2) % N) + 1)
        else:
            out_ref[sub_t, :, :CH] = (buf_r[s, :, :CH]
                                      + nxt_r[:, :CH]).astype(BF)
            out_ref[sub_b, :, :CH] = (buf_l[s, :, :CH]
                                      + nxt_l[:, :CH]).astype(BF)
        ry.wait_recv()
        ly.wait_recv()
        if s < N - 2:
            buf_r[s, :, CH:] = (buf_r[s, :, CH:] + nxt_r[:, CH:]).astype(BF)
            buf_l[s, :, CH:] = (buf_l[s, :, CH:] + nxt_l[:, CH:]).astype(BF)
            ry = rs_start(buf_r, 1, s + 1, right, rs_r_s, rs_r_r)
            ly = rs_start(buf_l, 1, s + 1, left, rs_l_s, rs_l_r)
            nxt_r, nxt_l = nnr, nnl
        else:
            out_ref[sub_t, :, CH:] = (buf_r[s, :, CH:]
                                      + nxt_r[:, CH:]).astype(BF)
            out_ref[sub_b, :, CH:] = (buf_l[s, :, CH:]
                                      + nxt_l[:, CH:]).astype(BF)

    def PX(d):
        return d ^ 1

    def PY(d):
        return 4 * (d // 4) + (3 - d % 4)

    def PZ(d):
        return d ^ 4

    tstarts = (0, 1024, 2560)
    twidths = (1024, 1536, 1536)
    orders = ((PX, PY, PZ), (PY, PZ, PX), (PZ, PX, PY))
    held = [[i], [i], [i]]
    ctr = [0, 0, 0]
    for p in range(3):
        phase_descs = []
        for t in range(3):
            P = orders[t][p]
            prt = P(i)
            for d in held[t]:
                for sub in (2 * ((d + 1) % N), 2 * ((d - 1) % N) + 1):
                    piece = out_ref.at[sub, :, pl.ds(tstarts[t], twidths[t])]
                    rd = pltpu.make_async_remote_copy(
                        src_ref=piece, dst_ref=piece,
                        send_sem=ag_s.at[t, ctr[t]],
                        recv_sem=ag_r.at[t, ctr[t]],
                        device_id=(prt,),
                        device_id_type=pl.DeviceIdType.MESH)
                    rd.start()
                    ctr[t] += 1
                    phase_descs.append(rd)
                    pending.append(rd)
            held[t] = held[t] + [P(d) for d in held[t]]
        for rd in phase_descs:
            rd.wait_recv()

    for rd in pending:
        rd.wait_send()


def _outar(q, kj, vj, qr, kr, wo_j):
    return pl.pallas_call(
        _outar_body,
        out_shape=jax.ShapeDtypeStruct((2 * N, RH, D), BF),
        in_specs=[pl.BlockSpec(memory_space=VMEM)] * 6,
        out_specs=pl.BlockSpec(memory_space=VMEM),
        scratch_shapes=[
            VMEM((N, RH, D), BF),
            VMEM((N, RH, D), BF),
            pltpu.SemaphoreType.DMA((2, N - 1)),
            pltpu.SemaphoreType.DMA((2, N - 1)),
            pltpu.SemaphoreType.DMA((2, N - 1)),
            pltpu.SemaphoreType.DMA((2, N - 1)),
            pltpu.SemaphoreType.DMA((3, 14)),
            pltpu.SemaphoreType.DMA((3, 14)),
        ],
        compiler_params=pltpu.CompilerParams(collective_id=1),
    )(q, kj, vj, qr, kr, wo_j)


def kernel(x, Wdkv, Wuk, Wuv, Wq, Wqr, Wkr, Wo):
    i = lax.axis_index("i")
    blk = (i + 1) % N

    xb = x.reshape(M, D).astype(BF)
    dc = Wdkv.shape[1]
    wukb = Wuk.astype(BF).reshape(dc, N, KC).transpose(1, 0, 2)
    wuvb = Wuv.astype(BF).reshape(dc, N, KC).transpose(1, 0, 2)

    wq_j = lax.dynamic_slice(Wq, (0, blk * KC), (D, KC)).astype(BF)
    wqr_j = lax.dynamic_slice(Wqr, (0, blk * QRC), (D, QRC)).astype(BF)
    wo_j = lax.dynamic_slice(Wo, (blk * KC, 0), (KC, D)).astype(BF)
    q, qr, kr, kj, vj = _kvq(xb, Wdkv.astype(BF), wukb, wuvb,
                             wq_j, wqr_j, Wkr.astype(BF))

    out16 = _outar(q, kj, vj, qr, kr, wo_j)
    return out16.reshape(M, D).reshape(B, S, D)


# device time: 159113 ns/iter; 2.7797x vs baseline; 1.0234x over previous
import jax
import jax.numpy as jnp
from jax import lax
from jax.experimental import pallas as pl
from jax.experimental.pallas import tpu as pltpu

N = 8
B, S, D = 4, 256, 4096
H, Dh, Dr = 32, 128, 64
HL = H // N
KC = HL * Dh
QRC = HL * Dr
M = B * S
RC = M // N
SCALE = (Dh + Dr) ** -0.5
BF = jnp.bfloat16
VMEM = pltpu.VMEM


def _kvq_body(x_ref, wdkv_ref, wukb_ref, wuvb_ref, wq_ref, wqr_ref, wkr_ref,
              q_ref, qr_ref, kr_ref, kj_ref, vj_ref,
              c_s, call_s, wka_s, wva_s, cs, cr, wks, wkrv, wvs, wvrv):
    i = lax.axis_index("i")
    blk = (i + 1) % N

    c_s[...] = jnp.dot(x_ref[...], wdkv_ref[...],
                       preferred_element_type=jnp.float32).astype(BF)

    bsem = pltpu.get_barrier_semaphore()
    for d in range(1, N):
        pl.semaphore_signal(bsem, inc=1, device_id=((i + d) % N,),
                            device_id_type=pl.DeviceIdType.MESH)
    pl.semaphore_wait(bsem, N - 1)

    dc = c_s.shape[1]
    descs = []
    for d in range(1, N):
        tgt = (i + d) % N
        blk_tgt = (tgt + 1) % N
        for src, dst, ssem, rsem in (
            (c_s, call_s.at[:, pl.ds((d - 1) * dc, dc)], cs, cr),
            (wukb_ref.at[blk_tgt],
             wka_s.at[pl.ds((d - 1) * dc, dc), :], wks, wkrv),
            (wuvb_ref.at[blk_tgt],
             wva_s.at[pl.ds((d - 1) * dc, dc), :], wvs, wvrv),
        ):
            rdma = pltpu.make_async_remote_copy(
                src_ref=src,
                dst_ref=dst,
                send_sem=ssem.at[d - 1],
                recv_sem=rsem.at[d - 1],
                device_id=(tgt,),
                device_id_type=pl.DeviceIdType.MESH,
            )
            rdma.start()
            descs.append(rdma)

    call_s[:, (N - 1) * dc:] = c_s[...]
    wka_s[(N - 1) * dc:, :] = wukb_ref[blk]
    wva_s[(N - 1) * dc:, :] = wuvb_ref[blk]
    x = x_ref[...]
    q_ref[...] = jnp.dot(x, wq_ref[...],
                         preferred_element_type=jnp.float32).astype(BF)
    qr_ref[...] = jnp.dot(x, wqr_ref[...],
                          preferred_element_type=jnp.float32).astype(BF)
    kr_ref[...] = jnp.dot(x, wkr_ref[...],
                          preferred_element_type=jnp.float32).astype(BF)

    for rdma in descs:
        rdma.wait()

    cfull = call_s[...]
    kj_ref[...] = jnp.dot(cfull, wka_s[...],
                          preferred_element_type=jnp.float32).astype(BF)
    vj_ref[...] = jnp.dot(cfull, wva_s[...],
                          preferred_element_type=jnp.float32).astype(BF)


def _kvq(xb, wdkv, wukb, wuvb, wq_j, wqr_j, wkr):
    dc = wdkv.shape[1]
    return pl.pallas_call(
        _kvq_body,
        out_shape=[jax.ShapeDtypeStruct((M, KC), BF),
                   jax.ShapeDtypeStruct((M, QRC), BF),
                   jax.ShapeDtypeStruct((M, Dr), BF),
                   jax.ShapeDtypeStruct((M, KC), BF),
                   jax.ShapeDtypeStruct((M, KC), BF)],
        in_specs=[pl.BlockSpec(memory_space=VMEM)] * 7,
        out_specs=[pl.BlockSpec(memory_space=VMEM)] * 5,
        scratch_shapes=[
            VMEM((M, dc), BF),
            VMEM((M, N * dc), BF),
            VMEM((N * dc, KC), BF),
            VMEM((N * dc, KC), BF),
            pltpu.SemaphoreType.DMA((N - 1,)),
            pltpu.SemaphoreType.DMA((N - 1,)),
            pltpu.SemaphoreType.DMA((N - 1,)),
            pltpu.SemaphoreType.DMA((N - 1,)),
            pltpu.SemaphoreType.DMA((N - 1,)),
            pltpu.SemaphoreType.DMA((N - 1,)),
        ],
        compiler_params=pltpu.CompilerParams(collective_id=0),
    )(xb, wdkv, wukb, wuvb, wq_j, wqr_j, wkr)


def _attn_body(q_ref, k_ref, v_ref, qr_ref, kr_ref, o_ref):
    kr = kr_ref[...]
    nums = (((1,), (1,)), ((), ()))
    for h in range(HL):
        hd = slice(h * Dh, (h + 1) * Dh)
        hr = slice(h * Dr, (h + 1) * Dr)
        s = (lax.dot_general(q_ref[:, hd], k_ref[:, hd], nums,
                             preferred_element_type=jnp.float32)
             + lax.dot_general(qr_ref[:, hr], kr, nums,
                               preferred_element_type=jnp.float32)) * SCALE
        m = jnp.max(s, axis=-1, keepdims=True)
        e = jnp.exp(s - m)
        p = (e / jnp.sum(e, axis=-1, keepdims=True)).astype(BF)
        o_ref[:, hd] = jnp.dot(p, v_ref[:, hd],
                               preferred_element_type=jnp.float32).astype(BF)


def _attention(q, kj, vj, qr, kr):
    return pl.pallas_call(
        _attn_body,
        grid=(B,),
        out_shape=jax.ShapeDtypeStruct((M, KC), BF),
        in_specs=[
            pl.BlockSpec((S, KC), lambda b: (b, 0)),
            pl.BlockSpec((S, KC), lambda b: (b, 0)),
            pl.BlockSpec((S, KC), lambda b: (b, 0)),
            pl.BlockSpec((S, QRC), lambda b: (b, 0)),
            pl.BlockSpec((S, Dr), lambda b: (b, 0)),
        ],
        out_specs=pl.BlockSpec((S, KC), lambda b: (b, 0)),
    )(q, kj, vj, qr, kr)


RH = RC // 2


def _outar_body(q_ref, kj_ref, vj_ref, qr_ref, kr_ref, wo_ref, out_ref,
                buf_r, buf_l,
                rs_r_s, rs_r_r, rs_l_s, rs_l_r,
                ag_s, ag_r):
    i = lax.axis_index("i")
    left = (i - 1) % N
    right = (i + 1) % N
    bsem = pltpu.get_barrier_semaphore()
    for nbr in (left, right):
        pl.semaphore_signal(bsem, inc=1, device_id=(nbr,),
                            device_id_type=pl.DeviceIdType.MESH)
    pl.semaphore_wait(bsem, 2)

    wo = wo_ref[...]
    nums = (((1,), (1,)), ((), ()))

    def part(sub):
        b0 = (sub // (S // RH)) * S
        qs = q_ref[pl.ds(sub * RH, RH), :]
        qrs = qr_ref[pl.ds(sub * RH, RH), :]
        kb = kj_ref[pl.ds(b0, S), :]
        vb = vj_ref[pl.ds(b0, S), :]
        krb = kr_ref[pl.ds(b0, S), :]
        ohs = []
        for h in range(HL):
            hd = slice(h * Dh, (h + 1) * Dh)
            hr = slice(h * Dr, (h + 1) * Dr)
            s = (lax.dot_general(qs[:, hd], kb[:, hd], nums,
                                 preferred_element_type=jnp.float32)
                 + lax.dot_general(qrs[:, hr], krb, nums,
                                   preferred_element_type=jnp.float32)) * SCALE
            m = jnp.max(s, axis=-1, keepdims=True)
            e = jnp.exp(s - m)
            p = (e / jnp.sum(e, axis=-1, keepdims=True)).astype(BF)
            ohs.append(jnp.dot(p, vb[:, hd],
                               preferred_element_type=jnp.float32).astype(BF))
        o_rows = jnp.concatenate(ohs, axis=1)
        return jnp.dot(o_rows, wo, preferred_element_type=jnp.float32)

    pending = []
    CH = D // 2
    sub_t = 2 * ((i + 1) % N)
    sub_b = 2 * ((i - 1) % N) + 1

    def rs_start(buf, ch, hop, dev, ssem, rsem):
        src = N - 1 if hop == 0 else hop - 1
        sl = pl.ds(ch * CH, CH)
        rd = pltpu.make_async_remote_copy(
            src_ref=buf.at[src, :, sl], dst_ref=buf.at[hop, :, sl],
            send_sem=ssem.at[ch, hop], recv_sem=rsem.at[ch, hop],
            device_id=(dev,), device_id_type=pl.DeviceIdType.MESH)
        rd.start()
        pending.append(rd)
        return rd

    buf_r[N - 1] = part(2 * i).astype(BF)
    buf_l[N - 1] = part(2 * i + 1).astype(BF)
    rx = rs_start(buf_r, 0, 0, right, rs_r_s, rs_r_r)
    lx = rs_start(buf_l, 0, 0, left, rs_l_s, rs_l_r)
    ry = rs_start(buf_r, 1, 0, right, rs_r_s, rs_r_r)
    ly = rs_start(buf_l, 1, 0, left, rs_l_s, rs_l_r)
    nxt_r = part(2 * ((i - 1) % N))
    nxt_l = part(2 * ((i + 1) % N) + 1)
    for s in range(N - 1):
        rx.wait_recv()
        lx.wait_recv()
        if s < N - 2:
            buf_r[s, :, :CH] = (buf_r[s, :, :CH] + nxt_r[:, :CH]).astype(BF)
            buf_l[s, :, :CH] = (buf_l[s, :, :CH] + nxt_l[:, :CH]).astype(BF)
            rx = rs_start(buf_r, 0, s + 1, right, rs_r_s, rs_r_r)
            lx = rs_start(buf_l, 0, s + 1, left, rs_l_s, rs_l_r)
            nnr = part(2 * ((i - s - 2) % N))
            nnl = part(2 * ((i + s + 2) % N) + 1)
        else:
            out_ref[sub_t, :, :CH] = (buf_r[s, :, :CH]
                                      + nxt_r[:, :CH]).astype(BF)
            out_ref[sub_b, :, :CH] = (buf_l[s, :, :CH]
                                      + nxt_l[:, :CH]).astype(BF)
        ry.wait_recv()
        ly.wait_recv()
        if s < N - 2:
            buf_r[s, :, CH:] = (buf_r[s, :, CH:] + nxt_r[:, CH:]).astype(BF)
            buf_l[s, :, CH:] = (buf_l[s, :, CH:] + nxt_l[:, CH:]).astype(BF)
            ry = rs_start(buf_r, 1, s + 1, right, rs_r_s, rs_r_r)
            ly = rs_start(buf_l, 1, s + 1, left, rs_l_s, rs_l_r)
            nxt_r, nxt_l = nnr, nnl
        else:
            out_ref[sub_t, :, CH:] = (buf_r[s, :, CH:]
                                      + nxt_r[:, CH:]).astype(BF)
            out_ref[sub_b, :, CH:] = (buf_l[s, :, CH:]
                                      + nxt_l[:, CH:]).astype(BF)

    def PX(d):
        return d ^ 1

    def PY(d):
        return 4 * (d // 4) + (3 - d % 4)

    def PZ(d):
        return d ^ 4

    tstarts = (0, 1280, 2688)
    twidths = (1280, 1408, 1408)
    orders = ((PX, PY, PZ), (PY, PZ, PX), (PZ, PX, PY))
    held = [[i], [i], [i]]
    ctr = [0, 0, 0]
    for p in range(3):
        phase_descs = []
        for t in range(3):
            P = orders[t][p]
            prt = P(i)
            for d in held[t]:
                for sub in (2 * ((d + 1) % N), 2 * ((d - 1) % N) + 1):
                    piece = out_ref.at[sub, :, pl.ds(tstarts[t], twidths[t])]
                    rd = pltpu.make_async_remote_copy(
                        src_ref=piece, dst_ref=piece,
                        send_sem=ag_s.at[t, ctr[t]],
                        recv_sem=ag_r.at[t, ctr[t]],
                        device_id=(prt,),
                        device_id_type=pl.DeviceIdType.MESH)
                    rd.start()
                    ctr[t] += 1
                    phase_descs.append(rd)
                    pending.append(rd)
            held[t] = held[t] + [P(d) for d in held[t]]
        for rd in phase_descs:
            rd.wait_recv()

    for rd in pending:
        rd.wait_send()


def _outar(q, kj, vj, qr, kr, wo_j):
    return pl.pallas_call(
        _outar_body,
        out_shape=jax.ShapeDtypeStruct((2 * N, RH, D), BF),
        in_specs=[pl.BlockSpec(memory_space=VMEM)] * 6,
        out_specs=pl.BlockSpec(memory_space=VMEM),
        scratch_shapes=[
            VMEM((N, RH, D), BF),
            VMEM((N, RH, D), BF),
            pltpu.SemaphoreType.DMA((2, N - 1)),
            pltpu.SemaphoreType.DMA((2, N - 1)),
            pltpu.SemaphoreType.DMA((2, N - 1)),
            pltpu.SemaphoreType.DMA((2, N - 1)),
            pltpu.SemaphoreType.DMA((3, 14)),
            pltpu.SemaphoreType.DMA((3, 14)),
        ],
        compiler_params=pltpu.CompilerParams(collective_id=1),
    )(q, kj, vj, qr, kr, wo_j)


def kernel(x, Wdkv, Wuk, Wuv, Wq, Wqr, Wkr, Wo):
    i = lax.axis_index("i")
    blk = (i + 1) % N

    xb = x.reshape(M, D).astype(BF)
    dc = Wdkv.shape[1]
    wukb = Wuk.astype(BF).reshape(dc, N, KC).transpose(1, 0, 2)
    wuvb = Wuv.astype(BF).reshape(dc, N, KC).transpose(1, 0, 2)

    wq_j = lax.dynamic_slice(Wq, (0, blk * KC), (D, KC)).astype(BF)
    wqr_j = lax.dynamic_slice(Wqr, (0, blk * QRC), (D, QRC)).astype(BF)
    wo_j = lax.dynamic_slice(Wo, (blk * KC, 0), (KC, D)).astype(BF)
    q, qr, kr, kj, vj = _kvq(xb, Wdkv.astype(BF), wukb, wuvb,
                             wq_j, wqr_j, Wkr.astype(BF))

    out16 = _outar(q, kj, vj, qr, kr, wo_j)
    return out16.reshape(M, D).reshape(B, S, D)


# device time: 154429 ns/iter; 2.8640x vs baseline; 1.0303x over previous
import jax
import jax.numpy as jnp
from jax import lax
from jax.experimental import pallas as pl
from jax.experimental.pallas import tpu as pltpu

N = 8
B, S, D = 4, 256, 4096
H, Dh, Dr = 32, 128, 64
HL = H // N
KC = HL * Dh
QRC = HL * Dr
M = B * S
RC = M // N
SCALE = (Dh + Dr) ** -0.5
BF = jnp.bfloat16
VMEM = pltpu.VMEM


def _kvq_body(x_ref, wdkv_ref, wukb_ref, wuvb_ref, wq_ref, wqr_ref, wkr_ref,
              q_ref, qr_ref, kr_ref, kj_ref, vj_ref,
              c_s, call_s, wka_s, wva_s, cs, cr, wks, wkrv, wvs, wvrv):
    i = lax.axis_index("i")
    blk = (i + 1) % N

    c_s[...] = jnp.dot(x_ref[...], wdkv_ref[...],
                       preferred_element_type=jnp.float32).astype(BF)

    bsem = pltpu.get_barrier_semaphore()
    for d in range(1, N):
        pl.semaphore_signal(bsem, inc=1, device_id=((i + d) % N,),
                            device_id_type=pl.DeviceIdType.MESH)
    pl.semaphore_wait(bsem, N - 1)

    dc = c_s.shape[1]
    descs = []
    for d in range(1, N):
        tgt = (i + d) % N
        blk_tgt = (tgt + 1) % N
        for src, dst, ssem, rsem in (
            (c_s, call_s.at[:, pl.ds((d - 1) * dc, dc)], cs, cr),
            (wukb_ref.at[blk_tgt],
             wka_s.at[pl.ds((d - 1) * dc, dc), :], wks, wkrv),
            (wuvb_ref.at[blk_tgt],
             wva_s.at[pl.ds((d - 1) * dc, dc), :], wvs, wvrv),
        ):
            rdma = pltpu.make_async_remote_copy(
                src_ref=src,
                dst_ref=dst,
                send_sem=ssem.at[d - 1],
                recv_sem=rsem.at[d - 1],
                device_id=(tgt,),
                device_id_type=pl.DeviceIdType.MESH,
            )
            rdma.start()
            descs.append(rdma)

    call_s[:, (N - 1) * dc:] = c_s[...]
    wka_s[(N - 1) * dc:, :] = wukb_ref[blk]
    wva_s[(N - 1) * dc:, :] = wuvb_ref[blk]
    x = x_ref[...]
    q_ref[...] = jnp.dot(x, wq_ref[...],
                         preferred_element_type=jnp.float32).astype(BF)
    qr_ref[...] = jnp.dot(x, wqr_ref[...],
                          preferred_element_type=jnp.float32).astype(BF)
    kr_ref[...] = jnp.dot(x, wkr_ref[...],
                          preferred_element_type=jnp.float32).astype(BF)

    for rdma in descs:
        rdma.wait()

    cfull = call_s[...]
    kj_ref[...] = jnp.dot(cfull, wka_s[...],
                          preferred_element_type=jnp.float32).astype(BF)
    vj_ref[...] = jnp.dot(cfull, wva_s[...],
                          preferred_element_type=jnp.float32).astype(BF)


def _kvq(xb, wdkv, wukb, wuvb, wq_j, wqr_j, wkr):
    dc = wdkv.shape[1]
    return pl.pallas_call(
        _kvq_body,
        out_shape=[jax.ShapeDtypeStruct((M, KC), BF),
                   jax.ShapeDtypeStruct((M, QRC), BF),
                   jax.ShapeDtypeStruct((M, Dr), BF),
                   jax.ShapeDtypeStruct((M, KC), BF),
                   jax.ShapeDtypeStruct((M, KC), BF)],
        in_specs=[pl.BlockSpec(memory_space=VMEM)] * 7,
        out_specs=[pl.BlockSpec(memory_space=VMEM)] * 5,
        scratch_shapes=[
            VMEM((M, dc), BF),
            VMEM((M, N * dc), BF),
            VMEM((N * dc, KC), BF),
            VMEM((N * dc, KC), BF),
            pltpu.SemaphoreType.DMA((N - 1,)),
            pltpu.SemaphoreType.DMA((N - 1,)),
            pltpu.SemaphoreType.DMA((N - 1,)),
            pltpu.SemaphoreType.DMA((N - 1,)),
            pltpu.SemaphoreType.DMA((N - 1,)),
            pltpu.SemaphoreType.DMA((N - 1,)),
        ],
        compiler_params=pltpu.CompilerParams(collective_id=0),
    )(xb, wdkv, wukb, wuvb, wq_j, wqr_j, wkr)


def _attn_body(q_ref, k_ref, v_ref, qr_ref, kr_ref, o_ref):
    kr = kr_ref[...]
    nums = (((1,), (1,)), ((), ()))
    for h in range(HL):
        hd = slice(h * Dh, (h + 1) * Dh)
        hr = slice(h * Dr, (h + 1) * Dr)
        s = (lax.dot_general(q_ref[:, hd], k_ref[:, hd], nums,
                             preferred_element_type=jnp.float32)
             + lax.dot_general(qr_ref[:, hr], kr, nums,
                               preferred_element_type=jnp.float32)) * SCALE
        m = jnp.max(s, axis=-1, keepdims=True)
        e = jnp.exp(s - m)
        p = (e / jnp.sum(e, axis=-1, keepdims=True)).astype(BF)
        o_ref[:, hd] = jnp.dot(p, v_ref[:, hd],
                               preferred_element_type=jnp.float32).astype(BF)


def _attention(q, kj, vj, qr, kr):
    return pl.pallas_call(
        _attn_body,
        grid=(B,),
        out_shape=jax.ShapeDtypeStruct((M, KC), BF),
        in_specs=[
            pl.BlockSpec((S, KC), lambda b: (b, 0)),
            pl.BlockSpec((S, KC), lambda b: (b, 0)),
            pl.BlockSpec((S, KC), lambda b: (b, 0)),
            pl.BlockSpec((S, QRC), lambda b: (b, 0)),
            pl.BlockSpec((S, Dr), lambda b: (b, 0)),
        ],
        out_specs=pl.BlockSpec((S, KC), lambda b: (b, 0)),
    )(q, kj, vj, qr, kr)


RH = RC // 2


def _outar_body(q_ref, kj_ref, vj_ref, qr_ref, kr_ref, wo_ref, out_ref,
                buf_r, buf_l,
                rs_r_s, rs_r_r, rs_l_s, rs_l_r,
                ag_s, ag_r):
    i = lax.axis_index("i")
    left = (i - 1) % N
    right = (i + 1) % N
    bsem = pltpu.get_barrier_semaphore()
    for nbr in (left, right):
        pl.semaphore_signal(bsem, inc=1, device_id=(nbr,),
                            device_id_type=pl.DeviceIdType.MESH)
    pl.semaphore_wait(bsem, 2)

    wo = wo_ref[...]
    nums = (((1,), (1,)), ((), ()))

    def att_rows(sub):
        b0 = (sub // (S // RH)) * S
        qs = q_ref[pl.ds(sub * RH, RH), :]
        qrs = qr_ref[pl.ds(sub * RH, RH), :]
        kb = kj_ref[pl.ds(b0, S), :]
        vb = vj_ref[pl.ds(b0, S), :]
        krb = kr_ref[pl.ds(b0, S), :]
        ohs = []
        for h in range(HL):
            hd = slice(h * Dh, (h + 1) * Dh)
            hr = slice(h * Dr, (h + 1) * Dr)
            s = (lax.dot_general(qs[:, hd], kb[:, hd], nums,
                                 preferred_element_type=jnp.float32)
                 + lax.dot_general(qrs[:, hr], krb, nums,
                                   preferred_element_type=jnp.float32)) * SCALE
            m = jnp.max(s, axis=-1, keepdims=True)
            e = jnp.exp(s - m)
            p = (e / jnp.sum(e, axis=-1, keepdims=True)).astype(BF)
            ohs.append(jnp.dot(p, vb[:, hd],
                               preferred_element_type=jnp.float32).astype(BF))
        return jnp.concatenate(ohs, axis=1)

    def part2(sub_r, sub_l):
        o2 = jnp.concatenate([att_rows(sub_r), att_rows(sub_l)], axis=0)
        g = jnp.dot(o2, wo, preferred_element_type=jnp.float32)
        return g[:RH], g[RH:]

    pending = []
    CH = D // 2
    sub_t = 2 * ((i + 1) % N)
    sub_b = 2 * ((i - 1) % N) + 1

    def rs_start(buf, ch, hop, dev, ssem, rsem):
        src = N - 1 if hop == 0 else hop - 1
        sl = pl.ds(ch * CH, CH)
        rd = pltpu.make_async_remote_copy(
            src_ref=buf.at[src, :, sl], dst_ref=buf.at[hop, :, sl],
            send_sem=ssem.at[ch, hop], recv_sem=rsem.at[ch, hop],
            device_id=(dev,), device_id_type=pl.DeviceIdType.MESH)
        rd.start()
        pending.append(rd)
        return rd

    own_r, own_l = part2(2 * i, 2 * i + 1)
    buf_r[N - 1] = own_r.astype(BF)
    buf_l[N - 1] = own_l.astype(BF)
    rx = rs_start(buf_r, 0, 0, right, rs_r_s, rs_r_r)
    lx = rs_start(buf_l, 0, 0, left, rs_l_s, rs_l_r)
    ry = rs_start(buf_r, 1, 0, right, rs_r_s, rs_r_r)
    ly = rs_start(buf_l, 1, 0, left, rs_l_s, rs_l_r)
    nxt_r, nxt_l = part2(2 * ((i - 1) % N), 2 * ((i + 1) % N) + 1)
    for s in range(N - 1):
        rx.wait_recv()
        lx.wait_recv()
        if s < N - 2:
            buf_r[s, :, :CH] = (buf_r[s, :, :CH] + nxt_r[:, :CH]).astype(BF)
            buf_l[s, :, :CH] = (buf_l[s, :, :CH] + nxt_l[:, :CH]).astype(BF)
            rx = rs_start(buf_r, 0, s + 1, right, rs_r_s, rs_r_r)
            lx = rs_start(buf_l, 0, s + 1, left, rs_l_s, rs_l_r)
            nnr, nnl = part2(2 * ((i - s - 2) % N),
                             2 * ((i + s + 2) % N) + 1)
        else:
            out_ref[sub_t, :, :CH] = (buf_r[s, :, :CH]
                                      + nxt_r[:, :CH]).astype(BF)
            out_ref[sub_b, :, :CH] = (buf_l[s, :, :CH]
                                      + nxt_l[:, :CH]).astype(BF)
        ry.wait_recv()
        ly.wait_recv()
        if s < N - 2:
            buf_r[s, :, CH:] = (buf_r[s, :, CH:] + nxt_r[:, CH:]).astype(BF)
            buf_l[s, :, CH:] = (buf_l[s, :, CH:] + nxt_l[:, CH:]).astype(BF)
            ry = rs_start(buf_r, 1, s + 1, right, rs_r_s, rs_r_r)
            ly = rs_start(buf_l, 1, s + 1, left, rs_l_s, rs_l_r)
            nxt_r, nxt_l = nnr, nnl
        else:
            out_ref[sub_t, :, CH:] = (buf_r[s, :, CH:]
                                      + nxt_r[:, CH:]).astype(BF)
            out_ref[sub_b, :, CH:] = (buf_l[s, :, CH:]
                                      + nxt_l[:, CH:]).astype(BF)

    def PX(d):
        return d ^ 1

    def PY(d):
        return 4 * (d // 4) + (3 - d % 4)

    def PZ(d):
        return d ^ 4

    tstarts = (0, 1280, 2688)
    twidths = (1280, 1408, 1408)
    orders = ((PX, PY, PZ), (PY, PZ, PX), (PZ, PX, PY))
    held = [[i], [i], [i]]
    ctr = [0, 0, 0]
    for p in range(3):
        phase_descs = []
        for t in range(3):
            P = orders[t][p]
            prt = P(i)
            for d in held[t]:
                for sub in (2 * ((d + 1) % N), 2 * ((d - 1) % N) + 1):
                    piece = out_ref.at[sub, :, pl.ds(tstarts[t], twidths[t])]
                    rd = pltpu.make_async_remote_copy(
                        src_ref=piece, dst_ref=piece,
                        send_sem=ag_s.at[t, ctr[t]],
                        recv_sem=ag_r.at[t, ctr[t]],
                        device_id=(prt,),
                        device_id_type=pl.DeviceIdType.MESH)
                    rd.start()
                    ctr[t] += 1
                    phase_descs.append(rd)
                    pending.append(rd)
            held[t] = held[t] + [P(d) for d in held[t]]
        for rd in phase_descs:
            rd.wait_recv()

    for rd in pending:
        rd.wait_send()


def _outar(q, kj, vj, qr, kr, wo_j):
    return pl.pallas_call(
        _outar_body,
        out_shape=jax.ShapeDtypeStruct((2 * N, RH, D), BF),
        in_specs=[pl.BlockSpec(memory_space=VMEM)] * 6,
        out_specs=pl.BlockSpec(memory_space=VMEM),
        scratch_shapes=[
            VMEM((N, RH, D), BF),
            VMEM((N, RH, D), BF),
            pltpu.SemaphoreType.DMA((2, N - 1)),
            pltpu.SemaphoreType.DMA((2, N - 1)),
            pltpu.SemaphoreType.DMA((2, N - 1)),
            pltpu.SemaphoreType.DMA((2, N - 1)),
            pltpu.SemaphoreType.DMA((3, 14)),
            pltpu.SemaphoreType.DMA((3, 14)),
        ],
        compiler_params=pltpu.CompilerParams(collective_id=1),
    )(q, kj, vj, qr, kr, wo_j)


def kernel(x, Wdkv, Wuk, Wuv, Wq, Wqr, Wkr, Wo):
    i = lax.axis_index("i")
    blk = (i + 1) % N

    xb = x.reshape(M, D).astype(BF)
    dc = Wdkv.shape[1]
    wukb = Wuk.astype(BF).reshape(dc, N, KC).transpose(1, 0, 2)
    wuvb = Wuv.astype(BF).reshape(dc, N, KC).transpose(1, 0, 2)

    wq_j = lax.dynamic_slice(Wq, (0, blk * KC), (D, KC)).astype(BF)
    wqr_j = lax.dynamic_slice(Wqr, (0, blk * QRC), (D, QRC)).astype(BF)
    wo_j = lax.dynamic_slice(Wo, (blk * KC, 0), (KC, D)).astype(BF)
    q, qr, kr, kj, vj = _kvq(xb, Wdkv.astype(BF), wukb, wuvb,
                             wq_j, wqr_j, Wkr.astype(BF))

    out16 = _outar(q, kj, vj, qr, kr, wo_j)
    return out16.reshape(M, D).reshape(B, S, D)
